# baseline (speedup 1.0000x reference)
import functools

import jax
import jax.numpy as jnp
from jax import lax
from jax.experimental import pallas as pl
from jax.experimental.pallas import tpu as pltpu
from jax.experimental.pallas import tpu_sc as plsc

NN = 10000
EE = 160000
HH = 128
BB = 64
LL = 3
EPS_ = 1e-5

NC = 2
NS = 16
NW = NC * NS
EW = EE // NW
CA = 40
NCHA = EW // CA
NP = 10240
SR = NP // NW
SRP = SR + 8
RCC = 2000
RNCH = EE // RCC
RING = 2080
ECAP = 81 * 2048
EPAD = EE + 8
TRASH = SR
SENT = EE * 512 + TRASH
RB = 640


def _prep_h_body(x_ref, wn_ref, bn_ref, wd_ref, ws_ref, h_ref, dt_ref, st_ref):
    h = jnp.maximum(
        jnp.dot(x_ref[...], wn_ref[...], preferred_element_type=jnp.float32)
        + bn_ref[...], 0.0)
    h_ref[...] = h
    dt_ref[...] = jnp.dot(h, wd_ref[...], preferred_element_type=jnp.float32)
    st_ref[...] = jnp.dot(h, ws_ref[...], preferred_element_type=jnp.float32)


def _prep_h(x, wn, bn, wd, ws):
    nb = NP // RB
    return pl.pallas_call(
        _prep_h_body,
        grid=(nb,),
        in_specs=[
            pl.BlockSpec((RB, HH), lambda i: (i, 0)),
            pl.BlockSpec((HH, HH), lambda i: (0, 0)),
            pl.BlockSpec((1, HH), lambda i: (0, 0)),
            pl.BlockSpec((HH, 2 * HH), lambda i: (0, 0)),
            pl.BlockSpec((HH, 2 * HH), lambda i: (0, 0)),
        ],
        out_specs=[
            pl.BlockSpec((RB, HH), lambda i: (i, 0)),
            pl.BlockSpec((RB, 2 * HH), lambda i: (i, 0)),
            pl.BlockSpec((RB, 2 * HH), lambda i: (i, 0)),
        ],
        out_shape=[
            jax.ShapeDtypeStruct((NP, HH), jnp.float32),
            jax.ShapeDtypeStruct((NP, 2 * HH), jnp.float32),
            jax.ShapeDtypeStruct((NP, 2 * HH), jnp.float32),
        ],
    )(x, wn, bn, wd, ws)


def _prep_q_body(ea_ref, wpe_ref, bpe_ref, wq_ref, bq_ref, *q_refs):
    e = jnp.maximum(
        jnp.dot(ea_ref[...], wpe_ref[...], preferred_element_type=jnp.float32)
        + bpe_ref[...], 0.0)
    bq = bq_ref[...]
    for l in range(LL):
        q_refs[l][...] = (
            jnp.dot(e, wq_ref[l], preferred_element_type=jnp.float32)
            + bq[l:l + 1, :])


def _prep_q(ea, wpe, bpe, wq, bq):
    nb = EE // RB
    return pl.pallas_call(
        _prep_q_body,
        grid=(nb,),
        in_specs=[
            pl.BlockSpec((RB, 16), lambda i: (i, 0)),
            pl.BlockSpec((16, HH), lambda i: (0, 0)),
            pl.BlockSpec((1, HH), lambda i: (0, 0)),
            pl.BlockSpec((LL, HH, 2 * HH), lambda i: (0, 0, 0)),
            pl.BlockSpec((LL, 2 * HH), lambda i: (0, 0)),
        ],
        out_specs=[pl.BlockSpec((RB, 2 * HH), lambda i: (i, 0))] * LL,
        out_shape=[jax.ShapeDtypeStruct((EE, 2 * HH), jnp.float32)] * LL,
    )(ea, wpe, bpe, wq, bq)


def _stats_body(agg_ref, mean_ref, rstd_ref, acc1, acc2):
    i = pl.program_id(0)

    @pl.when(i == 0)
    def _():
        acc1[...] = jnp.zeros_like(acc1)
        acc2[...] = jnp.zeros_like(acc2)

    a = agg_ref[...]
    acc1[...] += jnp.sum(a, axis=0, keepdims=True)
    acc2[...] += jnp.sum(a * a, axis=0, keepdims=True)

    @pl.when(i == pl.num_programs(0) - 1)
    def _():
        mu = acc1[...] * (1.0 / NN)
        var = acc2[...] * (1.0 / NN) - mu * mu
        mean_ref[...] = mu
        rstd_ref[...] = lax.rsqrt(var + EPS_)


def _stats(agg):
    nb = NP // RB
    return pl.pallas_call(
        _stats_body,
        grid=(nb,),
        in_specs=[pl.BlockSpec((RB, HH), lambda i: (i, 0))],
        out_specs=[
            pl.BlockSpec((1, HH), lambda i: (0, 0)),
            pl.BlockSpec((1, HH), lambda i: (0, 0)),
        ],
        out_shape=[
            jax.ShapeDtypeStruct((1, HH), jnp.float32),
            jax.ShapeDtypeStruct((1, HH), jnp.float32),
        ],
        scratch_shapes=[pltpu.VMEM((1, HH), jnp.float32)] * 2,
    )(agg)


def _bnproj_body(agg_ref, h_ref, mean_ref, rstd_ref, gam_ref, bet_ref,
                 wd_ref, ws_ref, hn_ref, dt_ref, st_ref):
    hn = ((agg_ref[...] - mean_ref[...]) * rstd_ref[...] * gam_ref[...]
          + bet_ref[...] + h_ref[...])
    hn_ref[...] = hn
    dt_ref[...] = jnp.dot(hn, wd_ref[...], preferred_element_type=jnp.float32)
    st_ref[...] = jnp.dot(hn, ws_ref[...], preferred_element_type=jnp.float32)


def _bnproj(agg, h, mean, rstd, gam, bet, wd, ws):
    nb = NP // RB
    return pl.pallas_call(
        _bnproj_body,
        grid=(nb,),
        in_specs=[
            pl.BlockSpec((RB, HH), lambda i: (i, 0)),
            pl.BlockSpec((RB, HH), lambda i: (i, 0)),
            pl.BlockSpec((1, HH), lambda i: (0, 0)),
            pl.BlockSpec((1, HH), lambda i: (0, 0)),
            pl.BlockSpec((1, HH), lambda i: (0, 0)),
            pl.BlockSpec((1, HH), lambda i: (0, 0)),
            pl.BlockSpec((HH, 2 * HH), lambda i: (0, 0)),
            pl.BlockSpec((HH, 2 * HH), lambda i: (0, 0)),
        ],
        out_specs=[
            pl.BlockSpec((RB, HH), lambda i: (i, 0)),
            pl.BlockSpec((RB, 2 * HH), lambda i: (i, 0)),
            pl.BlockSpec((RB, 2 * HH), lambda i: (i, 0)),
        ],
        out_shape=[
            jax.ShapeDtypeStruct((NP, HH), jnp.float32),
            jax.ShapeDtypeStruct((NP, 2 * HH), jnp.float32),
            jax.ShapeDtypeStruct((NP, 2 * HH), jnp.float32),
        ],
    )(agg, h, mean, rstd, gam, bet, wd, ws)


def _bnfinal_body(agg_ref, h_ref, mean_ref, rstd_ref, gam_ref, bet_ref,
                  wpp_ref, bpp_ref, h2_ref):
    hn = ((agg_ref[...] - mean_ref[...]) * rstd_ref[...] * gam_ref[...]
          + bet_ref[...] + h_ref[...])
    h2_ref[...] = jnp.maximum(
        jnp.dot(hn, wpp_ref[...], preferred_element_type=jnp.float32)
        + bpp_ref[...], 0.0)


def _bnfinal(agg, h, mean, rstd, gam, bet, wpp, bpp):
    nb = NP // RB
    return pl.pallas_call(
        _bnfinal_body,
        grid=(nb,),
        in_specs=[
            pl.BlockSpec((RB, HH), lambda i: (i, 0)),
            pl.BlockSpec((RB, HH), lambda i: (i, 0)),
            pl.BlockSpec((1, HH), lambda i: (0, 0)),
            pl.BlockSpec((1, HH), lambda i: (0, 0)),
            pl.BlockSpec((1, HH), lambda i: (0, 0)),
            pl.BlockSpec((1, HH), lambda i: (0, 0)),
            pl.BlockSpec((HH, HH), lambda i: (0, 0)),
            pl.BlockSpec((1, HH), lambda i: (0, 0)),
        ],
        out_specs=pl.BlockSpec((RB, HH), lambda i: (i, 0)),
        out_shape=jax.ShapeDtypeStruct((NP, HH), jnp.float32),
    )(agg, h, mean, rstd, gam, bet, wpp, bpp)


def _post_body(h2_ref, bat_ref, wp_ref, bp_ref, wo_ref, bo_ref, out_ref,
               pooled, cnt):
    i = pl.program_id(0)

    @pl.when(i == 0)
    def _():
        pooled[...] = jnp.zeros_like(pooled)
        cnt[...] = jnp.zeros_like(cnt)

    bat = bat_ref[...].reshape(1, RB)
    rows = lax.broadcasted_iota(jnp.int32, (BB, RB), 0)
    oh = (rows == bat).astype(jnp.float32)
    pooled[...] += jnp.dot(oh, h2_ref[...], preferred_element_type=jnp.float32)
    cnt[...] += jnp.broadcast_to(jnp.sum(oh, axis=1, keepdims=True), (BB, HH))

    @pl.when(i == pl.num_programs(0) - 1)
    def _():
        g = pooled[...] / jnp.maximum(cnt[...], 1.0)
        bp = bp_ref[...]
        for k in range(2):
            g = jnp.maximum(
                jnp.dot(g, wp_ref[k], preferred_element_type=jnp.float32)
                + bp[k:k + 1, :], 0.0)
        out_ref[...] = (jnp.dot(g, wo_ref[...], preferred_element_type=jnp.float32)
                        + bo_ref[...])


def _post(h2, bat3d, wp, bp, wo_pad, bo_pad):
    nb = NP // RB
    return pl.pallas_call(
        _post_body,
        grid=(nb,),
        in_specs=[
            pl.BlockSpec((RB, HH), lambda i: (i, 0)),
            pl.BlockSpec((1, 1, RB), lambda i: (i, 0, 0)),
            pl.BlockSpec((2, HH, HH), lambda i: (0, 0, 0)),
            pl.BlockSpec((2, HH), lambda i: (0, 0)),
            pl.BlockSpec((HH, HH), lambda i: (0, 0)),
            pl.BlockSpec((1, HH), lambda i: (0, 0)),
        ],
        out_specs=pl.BlockSpec((BB, HH), lambda i: (0, 0)),
        out_shape=jax.ShapeDtypeStruct((BB, HH), jnp.float32),
        scratch_shapes=[pltpu.VMEM((BB, HH), jnp.float32)] * 2,
    )(h2, bat3d, wp, bp, wo_pad, bo_pad)


def _sc_route_body(dst_hbm, list_hbm, cnt_hbm, ring_v, dchunk_v, cbuf_v, sem0):
    c = lax.axis_index("c")
    s = lax.axis_index("s")
    w = c * NS + s
    lo = w * SR
    hi = lo + SR
    iota16 = lax.iota(jnp.int32, 16)
    sent16 = jnp.full((16,), SENT, jnp.int32)

    def chunk(j, carry):
        pltpu.sync_copy(dst_hbm.at[pl.ds(j * RCC, RCC)], dchunk_v)

        def group(g, gcarry):
            goff, gbc = gcarry
            dv = dchunk_v[pl.ds(g * 16, 16)]
            eids = j * RCC + g * 16 + iota16
            m = jnp.logical_and(dv >= lo, dv < hi)
            packed = jnp.where(m, eids * 512 + (dv - lo), sent16)
            mi = jnp.where(m, 1, 0)
            for k in range(16):
                ring_v[pl.ds(goff, 16)] = jnp.full((16,), packed[k], jnp.int32)
                goff = goff + mi[k]

            def flush(args):
                foff, fbc = args
                pltpu.sync_copy(ring_v.at[pl.ds(0, 2048)],
                                list_hbm.at[w, pl.ds(fbc * 2048, 2048)])
                tailv = ring_v[pl.ds(2048, 16)]
                ring_v[pl.ds(0, 16)] = tailv
                return foff - 2048, fbc + 1

            return lax.cond(goff >= 2048, flush, lambda a: a, (goff, gbc))

        return lax.fori_loop(0, RCC // 16, group, carry)

    off, bc = lax.fori_loop(0, RNCH, chunk, (jnp.int32(0), jnp.int32(0)))
    ring_v[pl.ds(off, 16)] = sent16
    off = ((off + 15) // 16) * 16
    pltpu.sync_copy(ring_v.at[pl.ds(0, 2048)],
                    list_hbm.at[w, pl.ds(bc * 2048, 2048)])
    cbuf_v[pl.ds(0, 16)] = jnp.full((16,), bc * 2048 + off, jnp.int32)
    pltpu.sync_copy(cbuf_v, cnt_hbm.at[w])


@functools.lru_cache(maxsize=1)
def _get_sc_route():
  return pl.kernel(
    _sc_route_body,
    out_type=(
        jax.ShapeDtypeStruct((NW, ECAP), jnp.int32),
        jax.ShapeDtypeStruct((NW, 16), jnp.int32),
    ),
    mesh=plsc.VectorSubcoreMesh(core_axis_name="c", subcore_axis_name="s",
                                num_cores=NC, num_subcores=NS),
    scratch_types=[
        pltpu.VMEM((RING,), jnp.int32),
        pltpu.VMEM((RCC,), jnp.int32),
        pltpu.VMEM((16,), jnp.int32),
        pltpu.SemaphoreType.DMA,
    ],
  )


def _sc_msg_body(dst_hbm, src_hbm, dtbl_hbm, stbl_hbm, q_hbm, msg_hbm,
                 idxd_v, idxs_v, ab_v, gd_v, gs_v, msg_v, sem0, sem1):
    c = lax.axis_index("c")
    s = lax.axis_index("s")
    w = c * NS + s

    third = jnp.float32(1.0 / 3.0)
    fifth = jnp.float32(1.0 / 5.0)
    seventh = jnp.float32(1.0 / 7.0)

    def chunk(j, carry):
        base = w * EW + j * CA
        pltpu.sync_copy(dst_hbm.at[pl.ds(base, CA)], idxd_v)
        pltpu.sync_copy(src_hbm.at[pl.ds(base, CA)], idxs_v)
        pltpu.sync_copy(q_hbm.at[pl.ds(base, CA)], ab_v)
        pltpu.async_copy(dtbl_hbm.at[idxd_v], gd_v, sem0).wait()
        pltpu.async_copy(stbl_hbm.at[idxs_v], gs_v, sem1).wait()

        def edge(e, ecarry):
            for g in range(HH // 16):
                sl = pl.ds(g * 16, 16)
                slc = pl.ds(HH + g * 16, 16)
                a = ab_v[e, sl] + gd_v[e, sl] + gs_v[e, sl]
                b = ab_v[e, slc] + gd_v[e, slc] + gs_v[e, slc]
                ta = jnp.exp(-jnp.abs(a))
                u = ta / (2.0 + ta)
                u2 = u * u
                poly = 1.0 + u2 * (third + u2 * (fifth + u2 * seventh))
                sp = jnp.maximum(a, 0.0) + 2.0 * u * poly
                tb = jnp.exp(-jnp.abs(b))
                sg = jnp.where(b >= 0.0, 1.0, tb) / (1.0 + tb)
                msg_v[e, sl] = sp * sg
            return ecarry

        lax.fori_loop(0, CA, edge, 0)
        pltpu.sync_copy(msg_v, msg_hbm.at[pl.ds(base, CA)])
        return carry

    lax.fori_loop(0, NCHA, chunk, 0)


@functools.lru_cache(maxsize=1)
def _get_sc_msg():
  return pl.kernel(
    _sc_msg_body,
    out_type=jax.ShapeDtypeStruct((EPAD, HH), jnp.float32),
    mesh=plsc.VectorSubcoreMesh(core_axis_name="c", subcore_axis_name="s",
                                num_cores=NC, num_subcores=NS),
    scratch_types=[
        pltpu.VMEM((CA,), jnp.int32),
        pltpu.VMEM((CA,), jnp.int32),
        pltpu.VMEM((CA, 2 * HH), jnp.float32),
        pltpu.VMEM((CA, 2 * HH), jnp.float32),
        pltpu.VMEM((CA, 2 * HH), jnp.float32),
        pltpu.VMEM((CA, HH), jnp.float32),
        pltpu.SemaphoreType.DMA,
        pltpu.SemaphoreType.DMA,
    ],
  )


def _sc_agg_body(list_hbm, cnt_hbm, msg_hbm, agg_hbm,
                 agg_v, lbuf_v, rows_v, cbuf_v, sem0):
    c = lax.axis_index("c")
    s = lax.axis_index("s")
    w = c * NS + s
    zeros16 = jnp.zeros((16,), jnp.float32)

    def zr(i, carry):
        agg_v[pl.ds(i * 16, 16)] = zeros16
        return carry

    lax.fori_loop(0, SRP * HH // 16, zr, 0)

    pltpu.sync_copy(cnt_hbm.at[w], cbuf_v)
    cnt = cbuf_v[pl.ds(0, 16)][0]
    nblk = (cnt + 2047) // 2048

    def block(b, carry):
        pltpu.sync_copy(list_hbm.at[w, pl.ds(b * 2048, 2048)], lbuf_v)
        ngrp = jnp.minimum(128, (cnt - b * 2048 + 15) // 16)

        def group(g, gcarry):
            lv = lbuf_v[pl.ds(g * 16, 16)]
            eids = lax.shift_right_logical(lv, 9)
            dls = jnp.bitwise_and(lv, 511)
            pltpu.async_copy(msg_hbm.at[eids], rows_v, sem0).wait()
            for k in range(16):
                dl = dls[k]
                for g8 in range(HH // 16):
                    sl = pl.ds(dl * HH + g8 * 16, 16)
                    agg_v[sl] = agg_v[sl] + rows_v[k, pl.ds(g8 * 16, 16)]
            return gcarry

        lax.fori_loop(0, ngrp, group, 0)
        return carry

    lax.fori_loop(0, nblk, block, 0)

    def cpo(t, carry):
        for r in range(16):
            row = t * 16 + r
            for g8 in range(HH // 16):
                rows_v[r, pl.ds(g8 * 16, 16)] = agg_v[pl.ds(row * HH + g8 * 16, 16)]
        pltpu.sync_copy(rows_v, agg_hbm.at[pl.ds(w * SR + t * 16, 16)])
        return carry

    lax.fori_loop(0, SR // 16, cpo, 0)


@functools.lru_cache(maxsize=1)
def _get_sc_agg():
  return pl.kernel(
    _sc_agg_body,
    out_type=jax.ShapeDtypeStruct((NP, HH), jnp.float32),
    mesh=plsc.VectorSubcoreMesh(core_axis_name="c", subcore_axis_name="s",
                                num_cores=NC, num_subcores=NS),
    scratch_types=[
        pltpu.VMEM((SRP * HH,), jnp.float32),
        pltpu.VMEM((2048,), jnp.int32),
        pltpu.VMEM((16, HH), jnp.float32),
        pltpu.VMEM((16,), jnp.int32),
        pltpu.SemaphoreType.DMA,
    ],
  )


def kernel(x, edge_index, edge_attr, batch, W_pre_node, b_pre_node,
           W_pre_edge, b_pre_edge, W_soft, b_soft, W_conv, b_conv, gamma,
           beta, W_prepool, b_prepool, W_post, b_post, W_out, b_out):
    dst = edge_index[1]
    src = edge_index[0]

    wd = [jnp.concatenate([W_soft[l, :HH, :], W_conv[l, :HH, :]], axis=1)
          for l in range(LL)]
    ws = [jnp.concatenate([W_soft[l, HH:2 * HH, :], W_conv[l, HH:2 * HH, :]],
                          axis=1) for l in range(LL)]
    wq = jnp.stack([
        jnp.concatenate([W_soft[l, 2 * HH:, :], W_conv[l, 2 * HH:, :]], axis=1)
        for l in range(LL)])
    bq = jnp.stack([
        jnp.concatenate([b_soft[l], b_conv[l]]) for l in range(LL)])

    x_p = jnp.pad(x, ((0, NP - NN), (0, 0)))
    bat_p = jnp.pad(batch, (0, NP - NN), constant_values=BB)

    elist, ecnt = _get_sc_route()(dst)
    h, dtbl, stbl = _prep_h(x_p, W_pre_node, b_pre_node.reshape(1, HH),
                            wd[0], ws[0])
    qm = _prep_q(edge_attr, W_pre_edge, b_pre_edge.reshape(1, HH), wq, bq)

    h2 = None
    for l in range(LL):
        msg = _get_sc_msg()(dst, src, dtbl, stbl, qm[l])
        agg = _get_sc_agg()(elist, ecnt, msg)
        mean, rstd = _stats(agg)
        gam = gamma[l].reshape(1, HH)
        bet = beta[l].reshape(1, HH)
        if l < LL - 1:
            h, dtbl, stbl = _bnproj(agg, h, mean, rstd, gam, bet,
                                    wd[l + 1], ws[l + 1])
        else:
            h2 = _bnfinal(agg, h, mean, rstd, gam, bet, W_prepool,
                          b_prepool.reshape(1, HH))

    wo_pad = jnp.pad(W_out, ((0, 0), (0, HH - 1)))
    bo_pad = jnp.pad(b_out, (0, HH - 1)).reshape(1, HH)
    outp = _post(h2, bat_p.reshape(NP // RB, 1, RB), W_post, b_post,
                 wo_pad, bo_pad)
    return outp[:, 0]


# speedup vs baseline: 1.0308x; 1.0308x over previous
import functools

import jax
import jax.numpy as jnp
from jax import lax
from jax.experimental import pallas as pl
from jax.experimental.pallas import tpu as pltpu
from jax.experimental.pallas import tpu_sc as plsc

NN = 10000
EE = 160000
HH = 128
BB = 64
LL = 3
EPS_ = 1e-5

NC = 2
NS = 16
NW = NC * NS
EW = EE // NW
CA = 40
NCHA = EW // CA
NP = 10240
SR = NP // NW
SRP = SR + 8
RCC = 2000
RNCH = EE // RCC
RING = 2080
ECAP = 81 * 2048
EPAD = EE + 8
TRASH = SR
SENT = EE * 512 + TRASH
RB = 640


def _prep_h_body(x_ref, wn_ref, bn_ref, wd_ref, ws_ref, h_ref, dt_ref, st_ref):
    h = jnp.maximum(
        jnp.dot(x_ref[...], wn_ref[...], preferred_element_type=jnp.float32)
        + bn_ref[...], 0.0)
    h_ref[...] = h
    dt_ref[...] = jnp.dot(h, wd_ref[...], preferred_element_type=jnp.float32)
    st_ref[...] = jnp.dot(h, ws_ref[...], preferred_element_type=jnp.float32)


def _prep_h(x, wn, bn, wd, ws):
    nb = NP // RB
    return pl.pallas_call(
        _prep_h_body,
        grid=(nb,),
        in_specs=[
            pl.BlockSpec((RB, HH), lambda i: (i, 0)),
            pl.BlockSpec((HH, HH), lambda i: (0, 0)),
            pl.BlockSpec((1, HH), lambda i: (0, 0)),
            pl.BlockSpec((HH, 2 * HH), lambda i: (0, 0)),
            pl.BlockSpec((HH, 2 * HH), lambda i: (0, 0)),
        ],
        out_specs=[
            pl.BlockSpec((RB, HH), lambda i: (i, 0)),
            pl.BlockSpec((RB, 2 * HH), lambda i: (i, 0)),
            pl.BlockSpec((RB, 2 * HH), lambda i: (i, 0)),
        ],
        out_shape=[
            jax.ShapeDtypeStruct((NP, HH), jnp.float32),
            jax.ShapeDtypeStruct((NP, 2 * HH), jnp.float32),
            jax.ShapeDtypeStruct((NP, 2 * HH), jnp.float32),
        ],
    )(x, wn, bn, wd, ws)


def _prep_q_body(ea_ref, wpe_ref, bpe_ref, wq_ref, bq_ref, *q_refs):
    e = jnp.maximum(
        jnp.dot(ea_ref[...], wpe_ref[...], preferred_element_type=jnp.float32)
        + bpe_ref[...], 0.0)
    bq = bq_ref[...]
    for l in range(LL):
        q_refs[l][...] = (
            jnp.dot(e, wq_ref[l], preferred_element_type=jnp.float32)
            + bq[l:l + 1, :])


def _prep_q(ea, wpe, bpe, wq, bq):
    nb = EE // RB
    return pl.pallas_call(
        _prep_q_body,
        grid=(nb,),
        in_specs=[
            pl.BlockSpec((RB, 16), lambda i: (i, 0)),
            pl.BlockSpec((16, HH), lambda i: (0, 0)),
            pl.BlockSpec((1, HH), lambda i: (0, 0)),
            pl.BlockSpec((LL, HH, 2 * HH), lambda i: (0, 0, 0)),
            pl.BlockSpec((LL, 2 * HH), lambda i: (0, 0)),
        ],
        out_specs=[pl.BlockSpec((RB, 2 * HH), lambda i: (i, 0))] * LL,
        out_shape=[jax.ShapeDtypeStruct((EE, 2 * HH), jnp.float32)] * LL,
    )(ea, wpe, bpe, wq, bq)


def _stats_body(agg_ref, mean_ref, rstd_ref, acc1, acc2):
    i = pl.program_id(0)

    @pl.when(i == 0)
    def _():
        acc1[...] = jnp.zeros_like(acc1)
        acc2[...] = jnp.zeros_like(acc2)

    a = agg_ref[...]
    acc1[...] += jnp.sum(a, axis=0, keepdims=True)
    acc2[...] += jnp.sum(a * a, axis=0, keepdims=True)

    @pl.when(i == pl.num_programs(0) - 1)
    def _():
        mu = acc1[...] * (1.0 / NN)
        var = acc2[...] * (1.0 / NN) - mu * mu
        mean_ref[...] = mu
        rstd_ref[...] = lax.rsqrt(var + EPS_)


def _stats(agg):
    nb = NP // RB
    return pl.pallas_call(
        _stats_body,
        grid=(nb,),
        in_specs=[pl.BlockSpec((RB, HH), lambda i: (i, 0))],
        out_specs=[
            pl.BlockSpec((1, HH), lambda i: (0, 0)),
            pl.BlockSpec((1, HH), lambda i: (0, 0)),
        ],
        out_shape=[
            jax.ShapeDtypeStruct((1, HH), jnp.float32),
            jax.ShapeDtypeStruct((1, HH), jnp.float32),
        ],
        scratch_shapes=[pltpu.VMEM((1, HH), jnp.float32)] * 2,
    )(agg)


def _bnproj_body(agg_ref, h_ref, mean_ref, rstd_ref, gam_ref, bet_ref,
                 wd_ref, ws_ref, hn_ref, dt_ref, st_ref):
    hn = ((agg_ref[...] - mean_ref[...]) * rstd_ref[...] * gam_ref[...]
          + bet_ref[...] + h_ref[...])
    hn_ref[...] = hn
    dt_ref[...] = jnp.dot(hn, wd_ref[...], preferred_element_type=jnp.float32)
    st_ref[...] = jnp.dot(hn, ws_ref[...], preferred_element_type=jnp.float32)


def _bnproj(agg, h, mean, rstd, gam, bet, wd, ws):
    nb = NP // RB
    return pl.pallas_call(
        _bnproj_body,
        grid=(nb,),
        in_specs=[
            pl.BlockSpec((RB, HH), lambda i: (i, 0)),
            pl.BlockSpec((RB, HH), lambda i: (i, 0)),
            pl.BlockSpec((1, HH), lambda i: (0, 0)),
            pl.BlockSpec((1, HH), lambda i: (0, 0)),
            pl.BlockSpec((1, HH), lambda i: (0, 0)),
            pl.BlockSpec((1, HH), lambda i: (0, 0)),
            pl.BlockSpec((HH, 2 * HH), lambda i: (0, 0)),
            pl.BlockSpec((HH, 2 * HH), lambda i: (0, 0)),
        ],
        out_specs=[
            pl.BlockSpec((RB, HH), lambda i: (i, 0)),
            pl.BlockSpec((RB, 2 * HH), lambda i: (i, 0)),
            pl.BlockSpec((RB, 2 * HH), lambda i: (i, 0)),
        ],
        out_shape=[
            jax.ShapeDtypeStruct((NP, HH), jnp.float32),
            jax.ShapeDtypeStruct((NP, 2 * HH), jnp.float32),
            jax.ShapeDtypeStruct((NP, 2 * HH), jnp.float32),
        ],
    )(agg, h, mean, rstd, gam, bet, wd, ws)


def _bnfinal_body(agg_ref, h_ref, mean_ref, rstd_ref, gam_ref, bet_ref,
                  wpp_ref, bpp_ref, h2_ref):
    hn = ((agg_ref[...] - mean_ref[...]) * rstd_ref[...] * gam_ref[...]
          + bet_ref[...] + h_ref[...])
    h2_ref[...] = jnp.maximum(
        jnp.dot(hn, wpp_ref[...], preferred_element_type=jnp.float32)
        + bpp_ref[...], 0.0)


def _bnfinal(agg, h, mean, rstd, gam, bet, wpp, bpp):
    nb = NP // RB
    return pl.pallas_call(
        _bnfinal_body,
        grid=(nb,),
        in_specs=[
            pl.BlockSpec((RB, HH), lambda i: (i, 0)),
            pl.BlockSpec((RB, HH), lambda i: (i, 0)),
            pl.BlockSpec((1, HH), lambda i: (0, 0)),
            pl.BlockSpec((1, HH), lambda i: (0, 0)),
            pl.BlockSpec((1, HH), lambda i: (0, 0)),
            pl.BlockSpec((1, HH), lambda i: (0, 0)),
            pl.BlockSpec((HH, HH), lambda i: (0, 0)),
            pl.BlockSpec((1, HH), lambda i: (0, 0)),
        ],
        out_specs=pl.BlockSpec((RB, HH), lambda i: (i, 0)),
        out_shape=jax.ShapeDtypeStruct((NP, HH), jnp.float32),
    )(agg, h, mean, rstd, gam, bet, wpp, bpp)


def _post_body(h2_ref, bat_ref, wp_ref, bp_ref, wo_ref, bo_ref, out_ref,
               pooled, cnt):
    i = pl.program_id(0)

    @pl.when(i == 0)
    def _():
        pooled[...] = jnp.zeros_like(pooled)
        cnt[...] = jnp.zeros_like(cnt)

    bat = bat_ref[...].reshape(1, RB)
    rows = lax.broadcasted_iota(jnp.int32, (BB, RB), 0)
    oh = (rows == bat).astype(jnp.float32)
    pooled[...] += jnp.dot(oh, h2_ref[...], preferred_element_type=jnp.float32)
    cnt[...] += jnp.broadcast_to(jnp.sum(oh, axis=1, keepdims=True), (BB, HH))

    @pl.when(i == pl.num_programs(0) - 1)
    def _():
        g = pooled[...] / jnp.maximum(cnt[...], 1.0)
        bp = bp_ref[...]
        for k in range(2):
            g = jnp.maximum(
                jnp.dot(g, wp_ref[k], preferred_element_type=jnp.float32)
                + bp[k:k + 1, :], 0.0)
        out_ref[...] = (jnp.dot(g, wo_ref[...], preferred_element_type=jnp.float32)
                        + bo_ref[...])


def _post(h2, bat3d, wp, bp, wo_pad, bo_pad):
    nb = NP // RB
    return pl.pallas_call(
        _post_body,
        grid=(nb,),
        in_specs=[
            pl.BlockSpec((RB, HH), lambda i: (i, 0)),
            pl.BlockSpec((1, 1, RB), lambda i: (i, 0, 0)),
            pl.BlockSpec((2, HH, HH), lambda i: (0, 0, 0)),
            pl.BlockSpec((2, HH), lambda i: (0, 0)),
            pl.BlockSpec((HH, HH), lambda i: (0, 0)),
            pl.BlockSpec((1, HH), lambda i: (0, 0)),
        ],
        out_specs=pl.BlockSpec((BB, HH), lambda i: (0, 0)),
        out_shape=jax.ShapeDtypeStruct((BB, HH), jnp.float32),
        scratch_shapes=[pltpu.VMEM((BB, HH), jnp.float32)] * 2,
    )(h2, bat3d, wp, bp, wo_pad, bo_pad)


def _sc_route_body(dst_hbm, list_hbm, cnt_hbm, ring_v, dchunk_v, cbuf_v, sem0):
    c = lax.axis_index("c")
    s = lax.axis_index("s")
    w = c * NS + s
    lo = w * SR
    hi = lo + SR
    iota16 = lax.iota(jnp.int32, 16)
    sent16 = jnp.full((16,), SENT, jnp.int32)

    def chunk(j, carry):
        pltpu.sync_copy(dst_hbm.at[pl.ds(j * RCC, RCC)], dchunk_v)

        def group(g, gcarry):
            goff, gbc = gcarry
            dv = dchunk_v[pl.ds(g * 16, 16)]
            eids = j * RCC + g * 16 + iota16
            m = jnp.logical_and(dv >= lo, dv < hi)
            packed = jnp.where(m, eids * 512 + (dv - lo), sent16)
            mi = jnp.where(m, 1, 0)
            for k in range(16):
                ring_v[pl.ds(goff, 16)] = jnp.full((16,), packed[k], jnp.int32)
                goff = goff + mi[k]

            def flush(args):
                foff, fbc = args
                pltpu.sync_copy(ring_v.at[pl.ds(0, 2048)],
                                list_hbm.at[w, pl.ds(fbc * 2048, 2048)])
                tailv = ring_v[pl.ds(2048, 16)]
                ring_v[pl.ds(0, 16)] = tailv
                return foff - 2048, fbc + 1

            return lax.cond(goff >= 2048, flush, lambda a: a, (goff, gbc))

        return lax.fori_loop(0, RCC // 16, group, carry)

    off, bc = lax.fori_loop(0, RNCH, chunk, (jnp.int32(0), jnp.int32(0)))
    ring_v[pl.ds(off, 16)] = sent16
    off = ((off + 15) // 16) * 16
    pltpu.sync_copy(ring_v.at[pl.ds(0, 2048)],
                    list_hbm.at[w, pl.ds(bc * 2048, 2048)])
    cbuf_v[pl.ds(0, 16)] = jnp.full((16,), bc * 2048 + off, jnp.int32)
    pltpu.sync_copy(cbuf_v, cnt_hbm.at[w])


@functools.lru_cache(maxsize=1)
def _get_sc_route():
  return pl.kernel(
    _sc_route_body,
    out_type=(
        jax.ShapeDtypeStruct((NW, ECAP), jnp.int32),
        jax.ShapeDtypeStruct((NW, 16), jnp.int32),
    ),
    mesh=plsc.VectorSubcoreMesh(core_axis_name="c", subcore_axis_name="s",
                                num_cores=NC, num_subcores=NS),
    scratch_types=[
        pltpu.VMEM((RING,), jnp.int32),
        pltpu.VMEM((RCC,), jnp.int32),
        pltpu.VMEM((16,), jnp.int32),
        pltpu.SemaphoreType.DMA,
    ],
  )


def _sc_msg_body(dst_hbm, src_hbm, dtbl_hbm, stbl_hbm, q_hbm, msg_hbm,
                 idxd_v, idxs_v, ab_v, gd_v, gs_v, msg_v, sem0, sem1):
    c = lax.axis_index("c")
    s = lax.axis_index("s")
    w = c * NS + s

    third = jnp.float32(1.0 / 3.0)
    fifth = jnp.float32(1.0 / 5.0)
    seventh = jnp.float32(1.0 / 7.0)

    def chunk(j, carry):
        base = w * EW + j * CA
        pltpu.sync_copy(dst_hbm.at[pl.ds(base, CA)], idxd_v)
        pltpu.sync_copy(src_hbm.at[pl.ds(base, CA)], idxs_v)
        pltpu.sync_copy(q_hbm.at[pl.ds(base, CA)], ab_v)
        pltpu.async_copy(dtbl_hbm.at[idxd_v], gd_v, sem0).wait()
        pltpu.async_copy(stbl_hbm.at[idxs_v], gs_v, sem1).wait()

        @plsc.parallel_loop(0, CA, unroll=4)
        def edge(e):
            for g in range(HH // 16):
                sl = pl.ds(g * 16, 16)
                slc = pl.ds(HH + g * 16, 16)
                a = ab_v[e, sl] + gd_v[e, sl] + gs_v[e, sl]
                b = ab_v[e, slc] + gd_v[e, slc] + gs_v[e, slc]
                ta = jnp.exp(-jnp.abs(a))
                u = ta / (2.0 + ta)
                u2 = u * u
                poly = 1.0 + u2 * (third + u2 * (fifth + u2 * seventh))
                sp = jnp.maximum(a, 0.0) + 2.0 * u * poly
                tb = jnp.exp(-jnp.abs(b))
                sg = jnp.where(b >= 0.0, 1.0, tb) / (1.0 + tb)
                msg_v[e, sl] = sp * sg
        pltpu.sync_copy(msg_v, msg_hbm.at[pl.ds(base, CA)])
        return carry

    lax.fori_loop(0, NCHA, chunk, 0)


@functools.lru_cache(maxsize=1)
def _get_sc_msg():
  return pl.kernel(
    _sc_msg_body,
    out_type=jax.ShapeDtypeStruct((EPAD, HH), jnp.float32),
    mesh=plsc.VectorSubcoreMesh(core_axis_name="c", subcore_axis_name="s",
                                num_cores=NC, num_subcores=NS),
    scratch_types=[
        pltpu.VMEM((CA,), jnp.int32),
        pltpu.VMEM((CA,), jnp.int32),
        pltpu.VMEM((CA, 2 * HH), jnp.float32),
        pltpu.VMEM((CA, 2 * HH), jnp.float32),
        pltpu.VMEM((CA, 2 * HH), jnp.float32),
        pltpu.VMEM((CA, HH), jnp.float32),
        pltpu.SemaphoreType.DMA,
        pltpu.SemaphoreType.DMA,
    ],
  )


def _sc_agg_body(list_hbm, cnt_hbm, msg_hbm, agg_hbm,
                 agg_v, lbuf_v, rows_v, cbuf_v, sem0):
    c = lax.axis_index("c")
    s = lax.axis_index("s")
    w = c * NS + s
    zeros16 = jnp.zeros((16,), jnp.float32)

    def zr(i, carry):
        agg_v[pl.ds(i * 16, 16)] = zeros16
        return carry

    lax.fori_loop(0, SRP * HH // 16, zr, 0)

    pltpu.sync_copy(cnt_hbm.at[w], cbuf_v)
    cnt = cbuf_v[pl.ds(0, 16)][0]
    nblk = (cnt + 2047) // 2048

    def block(b, carry):
        pltpu.sync_copy(list_hbm.at[w, pl.ds(b * 2048, 2048)], lbuf_v)
        ngrp = jnp.minimum(128, (cnt - b * 2048 + 15) // 16)

        def group(g, gcarry):
            lv = lbuf_v[pl.ds(g * 16, 16)]
            eids = lax.shift_right_logical(lv, 9)
            dls = jnp.bitwise_and(lv, 511)
            pltpu.async_copy(msg_hbm.at[eids], rows_v, sem0).wait()
            for k in range(16):
                dl = dls[k]
                for g8 in range(HH // 16):
                    sl = pl.ds(dl * HH + g8 * 16, 16)
                    agg_v[sl] = agg_v[sl] + rows_v[k, pl.ds(g8 * 16, 16)]
            return gcarry

        lax.fori_loop(0, ngrp, group, 0)
        return carry

    lax.fori_loop(0, nblk, block, 0)

    def cpo(t, carry):
        for r in range(16):
            row = t * 16 + r
            for g8 in range(HH // 16):
                rows_v[r, pl.ds(g8 * 16, 16)] = agg_v[pl.ds(row * HH + g8 * 16, 16)]
        pltpu.sync_copy(rows_v, agg_hbm.at[pl.ds(w * SR + t * 16, 16)])
        return carry

    lax.fori_loop(0, SR // 16, cpo, 0)


@functools.lru_cache(maxsize=1)
def _get_sc_agg():
  return pl.kernel(
    _sc_agg_body,
    out_type=jax.ShapeDtypeStruct((NP, HH), jnp.float32),
    mesh=plsc.VectorSubcoreMesh(core_axis_name="c", subcore_axis_name="s",
                                num_cores=NC, num_subcores=NS),
    scratch_types=[
        pltpu.VMEM((SRP * HH,), jnp.float32),
        pltpu.VMEM((2048,), jnp.int32),
        pltpu.VMEM((16, HH), jnp.float32),
        pltpu.VMEM((16,), jnp.int32),
        pltpu.SemaphoreType.DMA,
    ],
  )


def kernel(x, edge_index, edge_attr, batch, W_pre_node, b_pre_node,
           W_pre_edge, b_pre_edge, W_soft, b_soft, W_conv, b_conv, gamma,
           beta, W_prepool, b_prepool, W_post, b_post, W_out, b_out):
    dst = edge_index[1]
    src = edge_index[0]

    wd = [jnp.concatenate([W_soft[l, :HH, :], W_conv[l, :HH, :]], axis=1)
          for l in range(LL)]
    ws = [jnp.concatenate([W_soft[l, HH:2 * HH, :], W_conv[l, HH:2 * HH, :]],
                          axis=1) for l in range(LL)]
    wq = jnp.stack([
        jnp.concatenate([W_soft[l, 2 * HH:, :], W_conv[l, 2 * HH:, :]], axis=1)
        for l in range(LL)])
    bq = jnp.stack([
        jnp.concatenate([b_soft[l], b_conv[l]]) for l in range(LL)])

    x_p = jnp.pad(x, ((0, NP - NN), (0, 0)))
    bat_p = jnp.pad(batch, (0, NP - NN), constant_values=BB)

    elist, ecnt = _get_sc_route()(dst)
    h, dtbl, stbl = _prep_h(x_p, W_pre_node, b_pre_node.reshape(1, HH),
                            wd[0], ws[0])
    qm = _prep_q(edge_attr, W_pre_edge, b_pre_edge.reshape(1, HH), wq, bq)

    h2 = None
    for l in range(LL):
        msg = _get_sc_msg()(dst, src, dtbl, stbl, qm[l])
        agg = _get_sc_agg()(elist, ecnt, msg)
        mean, rstd = _stats(agg)
        gam = gamma[l].reshape(1, HH)
        bet = beta[l].reshape(1, HH)
        if l < LL - 1:
            h, dtbl, stbl = _bnproj(agg, h, mean, rstd, gam, bet,
                                    wd[l + 1], ws[l + 1])
        else:
            h2 = _bnfinal(agg, h, mean, rstd, gam, bet, W_prepool,
                          b_prepool.reshape(1, HH))

    wo_pad = jnp.pad(W_out, ((0, 0), (0, HH - 1)))
    bo_pad = jnp.pad(b_out, (0, HH - 1)).reshape(1, HH)
    outp = _post(h2, bat_p.reshape(NP // RB, 1, RB), W_post, b_post,
                 wo_pad, bo_pad)
    return outp[:, 0]


# speedup vs baseline: 1.1285x; 1.0947x over previous
import functools

import jax
import jax.numpy as jnp
from jax import lax
from jax.experimental import pallas as pl
from jax.experimental.pallas import tpu as pltpu
from jax.experimental.pallas import tpu_sc as plsc

NN = 10000
EE = 160000
HH = 128
BB = 64
LL = 3
EPS_ = 1e-5

NC = 2
NS = 16
NW = NC * NS
EE2 = 163840
EW = EE2 // NW
CA = 128
NCHA = EW // CA
NP = 10240
SR = NP // NW
SRP = SR + 8
RCC = 2000
RNCH = EE // RCC
RING = 2080
ECAP = 81 * 2048
EPAD = EE2
TRASH = SR
SENT = EE * 512 + TRASH
RB = 640


def _prep_h_body(x_ref, wn_ref, bn_ref, wd_ref, ws_ref, h_ref, dt_ref, st_ref):
    h = jnp.maximum(
        jnp.dot(x_ref[...], wn_ref[...], preferred_element_type=jnp.float32)
        + bn_ref[...], 0.0)
    h_ref[...] = h
    dt_ref[...] = jnp.dot(h, wd_ref[...], preferred_element_type=jnp.float32)
    st_ref[...] = jnp.dot(h, ws_ref[...], preferred_element_type=jnp.float32)


def _prep_h(x, wn, bn, wd, ws):
    nb = NP // RB
    return pl.pallas_call(
        _prep_h_body,
        grid=(nb,),
        in_specs=[
            pl.BlockSpec((RB, HH), lambda i: (i, 0)),
            pl.BlockSpec((HH, HH), lambda i: (0, 0)),
            pl.BlockSpec((1, HH), lambda i: (0, 0)),
            pl.BlockSpec((HH, 2 * HH), lambda i: (0, 0)),
            pl.BlockSpec((HH, 2 * HH), lambda i: (0, 0)),
        ],
        out_specs=[
            pl.BlockSpec((RB, HH), lambda i: (i, 0)),
            pl.BlockSpec((RB, 2 * HH), lambda i: (i, 0)),
            pl.BlockSpec((RB, 2 * HH), lambda i: (i, 0)),
        ],
        out_shape=[
            jax.ShapeDtypeStruct((NP, HH), jnp.float32),
            jax.ShapeDtypeStruct((NP, 2 * HH), jnp.float32),
            jax.ShapeDtypeStruct((NP, 2 * HH), jnp.float32),
        ],
    )(x, wn, bn, wd, ws)


def _prep_q_body(ea_ref, wpe_ref, bpe_ref, wq_ref, bq_ref, *q_refs):
    e = jnp.maximum(
        jnp.dot(ea_ref[...], wpe_ref[...], preferred_element_type=jnp.float32)
        + bpe_ref[...], 0.0)
    bq = bq_ref[...]
    for l in range(LL):
        q_refs[l][...] = (
            jnp.dot(e, wq_ref[l], preferred_element_type=jnp.float32)
            + bq[l:l + 1, :])


def _prep_q(ea, wpe, bpe, wq, bq):
    nb = EE2 // RB
    return pl.pallas_call(
        _prep_q_body,
        grid=(nb,),
        in_specs=[
            pl.BlockSpec((RB, 16), lambda i: (i, 0)),
            pl.BlockSpec((16, HH), lambda i: (0, 0)),
            pl.BlockSpec((1, HH), lambda i: (0, 0)),
            pl.BlockSpec((LL, HH, 2 * HH), lambda i: (0, 0, 0)),
            pl.BlockSpec((LL, 2 * HH), lambda i: (0, 0)),
        ],
        out_specs=[pl.BlockSpec((RB, 2 * HH), lambda i: (i, 0))] * LL,
        out_shape=[jax.ShapeDtypeStruct((EE2, 2 * HH), jnp.float32)] * LL,
    )(ea, wpe, bpe, wq, bq)


def _stats_body(agg_ref, mean_ref, rstd_ref, acc1, acc2):
    i = pl.program_id(0)

    @pl.when(i == 0)
    def _():
        acc1[...] = jnp.zeros_like(acc1)
        acc2[...] = jnp.zeros_like(acc2)

    a = agg_ref[...]
    acc1[...] += jnp.sum(a, axis=0, keepdims=True)
    acc2[...] += jnp.sum(a * a, axis=0, keepdims=True)

    @pl.when(i == pl.num_programs(0) - 1)
    def _():
        mu = acc1[...] * (1.0 / NN)
        var = acc2[...] * (1.0 / NN) - mu * mu
        mean_ref[...] = mu
        rstd_ref[...] = lax.rsqrt(var + EPS_)


def _stats(agg):
    nb = NP // RB
    return pl.pallas_call(
        _stats_body,
        grid=(nb,),
        in_specs=[pl.BlockSpec((RB, HH), lambda i: (i, 0))],
        out_specs=[
            pl.BlockSpec((1, HH), lambda i: (0, 0)),
            pl.BlockSpec((1, HH), lambda i: (0, 0)),
        ],
        out_shape=[
            jax.ShapeDtypeStruct((1, HH), jnp.float32),
            jax.ShapeDtypeStruct((1, HH), jnp.float32),
        ],
        scratch_shapes=[pltpu.VMEM((1, HH), jnp.float32)] * 2,
    )(agg)


def _bnproj_body(agg_ref, h_ref, mean_ref, rstd_ref, gam_ref, bet_ref,
                 wd_ref, ws_ref, hn_ref, dt_ref, st_ref):
    hn = ((agg_ref[...] - mean_ref[...]) * rstd_ref[...] * gam_ref[...]
          + bet_ref[...] + h_ref[...])
    hn_ref[...] = hn
    dt_ref[...] = jnp.dot(hn, wd_ref[...], preferred_element_type=jnp.float32)
    st_ref[...] = jnp.dot(hn, ws_ref[...], preferred_element_type=jnp.float32)


def _bnproj(agg, h, mean, rstd, gam, bet, wd, ws):
    nb = NP // RB
    return pl.pallas_call(
        _bnproj_body,
        grid=(nb,),
        in_specs=[
            pl.BlockSpec((RB, HH), lambda i: (i, 0)),
            pl.BlockSpec((RB, HH), lambda i: (i, 0)),
            pl.BlockSpec((1, HH), lambda i: (0, 0)),
            pl.BlockSpec((1, HH), lambda i: (0, 0)),
            pl.BlockSpec((1, HH), lambda i: (0, 0)),
            pl.BlockSpec((1, HH), lambda i: (0, 0)),
            pl.BlockSpec((HH, 2 * HH), lambda i: (0, 0)),
            pl.BlockSpec((HH, 2 * HH), lambda i: (0, 0)),
        ],
        out_specs=[
            pl.BlockSpec((RB, HH), lambda i: (i, 0)),
            pl.BlockSpec((RB, 2 * HH), lambda i: (i, 0)),
            pl.BlockSpec((RB, 2 * HH), lambda i: (i, 0)),
        ],
        out_shape=[
            jax.ShapeDtypeStruct((NP, HH), jnp.float32),
            jax.ShapeDtypeStruct((NP, 2 * HH), jnp.float32),
            jax.ShapeDtypeStruct((NP, 2 * HH), jnp.float32),
        ],
    )(agg, h, mean, rstd, gam, bet, wd, ws)


def _bnfinal_body(agg_ref, h_ref, mean_ref, rstd_ref, gam_ref, bet_ref,
                  wpp_ref, bpp_ref, h2_ref):
    hn = ((agg_ref[...] - mean_ref[...]) * rstd_ref[...] * gam_ref[...]
          + bet_ref[...] + h_ref[...])
    h2_ref[...] = jnp.maximum(
        jnp.dot(hn, wpp_ref[...], preferred_element_type=jnp.float32)
        + bpp_ref[...], 0.0)


def _bnfinal(agg, h, mean, rstd, gam, bet, wpp, bpp):
    nb = NP // RB
    return pl.pallas_call(
        _bnfinal_body,
        grid=(nb,),
        in_specs=[
            pl.BlockSpec((RB, HH), lambda i: (i, 0)),
            pl.BlockSpec((RB, HH), lambda i: (i, 0)),
            pl.BlockSpec((1, HH), lambda i: (0, 0)),
            pl.BlockSpec((1, HH), lambda i: (0, 0)),
            pl.BlockSpec((1, HH), lambda i: (0, 0)),
            pl.BlockSpec((1, HH), lambda i: (0, 0)),
            pl.BlockSpec((HH, HH), lambda i: (0, 0)),
            pl.BlockSpec((1, HH), lambda i: (0, 0)),
        ],
        out_specs=pl.BlockSpec((RB, HH), lambda i: (i, 0)),
        out_shape=jax.ShapeDtypeStruct((NP, HH), jnp.float32),
    )(agg, h, mean, rstd, gam, bet, wpp, bpp)


def _post_body(h2_ref, bat_ref, wp_ref, bp_ref, wo_ref, bo_ref, out_ref,
               pooled, cnt):
    i = pl.program_id(0)

    @pl.when(i == 0)
    def _():
        pooled[...] = jnp.zeros_like(pooled)
        cnt[...] = jnp.zeros_like(cnt)

    bat = bat_ref[...].reshape(1, RB)
    rows = lax.broadcasted_iota(jnp.int32, (BB, RB), 0)
    oh = (rows == bat).astype(jnp.float32)
    pooled[...] += jnp.dot(oh, h2_ref[...], preferred_element_type=jnp.float32)
    cnt[...] += jnp.broadcast_to(jnp.sum(oh, axis=1, keepdims=True), (BB, HH))

    @pl.when(i == pl.num_programs(0) - 1)
    def _():
        g = pooled[...] / jnp.maximum(cnt[...], 1.0)
        bp = bp_ref[...]
        for k in range(2):
            g = jnp.maximum(
                jnp.dot(g, wp_ref[k], preferred_element_type=jnp.float32)
                + bp[k:k + 1, :], 0.0)
        out_ref[...] = (jnp.dot(g, wo_ref[...], preferred_element_type=jnp.float32)
                        + bo_ref[...])


def _post(h2, bat3d, wp, bp, wo_pad, bo_pad):
    nb = NP // RB
    return pl.pallas_call(
        _post_body,
        grid=(nb,),
        in_specs=[
            pl.BlockSpec((RB, HH), lambda i: (i, 0)),
            pl.BlockSpec((1, 1, RB), lambda i: (i, 0, 0)),
            pl.BlockSpec((2, HH, HH), lambda i: (0, 0, 0)),
            pl.BlockSpec((2, HH), lambda i: (0, 0)),
            pl.BlockSpec((HH, HH), lambda i: (0, 0)),
            pl.BlockSpec((1, HH), lambda i: (0, 0)),
        ],
        out_specs=pl.BlockSpec((BB, HH), lambda i: (0, 0)),
        out_shape=jax.ShapeDtypeStruct((BB, HH), jnp.float32),
        scratch_shapes=[pltpu.VMEM((BB, HH), jnp.float32)] * 2,
    )(h2, bat3d, wp, bp, wo_pad, bo_pad)


def _sc_route_body(dst_hbm, list_hbm, cnt_hbm, ring_v, dchunk_v, cbuf_v, sem0):
    c = lax.axis_index("c")
    s = lax.axis_index("s")
    w = c * NS + s
    lo = w * SR
    hi = lo + SR
    iota16 = lax.iota(jnp.int32, 16)
    sent16 = jnp.full((16,), SENT, jnp.int32)

    def chunk(j, carry):
        pltpu.sync_copy(dst_hbm.at[pl.ds(j * RCC, RCC)], dchunk_v)

        def group(g, gcarry):
            goff, gbc = gcarry
            dv = dchunk_v[pl.ds(g * 16, 16)]
            eids = j * RCC + g * 16 + iota16
            m = jnp.logical_and(dv >= lo, dv < hi)
            packed = jnp.where(m, eids * 512 + (dv - lo), sent16)
            mi = jnp.where(m, 1, 0)
            for k in range(16):
                ring_v[pl.ds(goff, 16)] = jnp.full((16,), packed[k], jnp.int32)
                goff = goff + mi[k]

            def flush(args):
                foff, fbc = args
                pltpu.sync_copy(ring_v.at[pl.ds(0, 2048)],
                                list_hbm.at[w, pl.ds(fbc * 2048, 2048)])
                tailv = ring_v[pl.ds(2048, 16)]
                ring_v[pl.ds(0, 16)] = tailv
                return foff - 2048, fbc + 1

            return lax.cond(goff >= 2048, flush, lambda a: a, (goff, gbc))

        return lax.fori_loop(0, RCC // 16, group, carry)

    off, bc = lax.fori_loop(0, RNCH, chunk, (jnp.int32(0), jnp.int32(0)))
    ring_v[pl.ds(off, 16)] = sent16
    off = ((off + 15) // 16) * 16
    pltpu.sync_copy(ring_v.at[pl.ds(0, 2048)],
                    list_hbm.at[w, pl.ds(bc * 2048, 2048)])
    cbuf_v[pl.ds(0, 16)] = jnp.full((16,), bc * 2048 + off, jnp.int32)
    pltpu.sync_copy(cbuf_v, cnt_hbm.at[w])


@functools.lru_cache(maxsize=1)
def _get_sc_route():
  return pl.kernel(
    _sc_route_body,
    out_type=(
        jax.ShapeDtypeStruct((NW, ECAP), jnp.int32),
        jax.ShapeDtypeStruct((NW, 16), jnp.int32),
    ),
    mesh=plsc.VectorSubcoreMesh(core_axis_name="c", subcore_axis_name="s",
                                num_cores=NC, num_subcores=NS),
    scratch_types=[
        pltpu.VMEM((RING,), jnp.int32),
        pltpu.VMEM((RCC,), jnp.int32),
        pltpu.VMEM((16,), jnp.int32),
        pltpu.SemaphoreType.DMA,
    ],
  )


def _sc_msg_body(dst_hbm, src_hbm, dtbl_hbm, stbl_hbm, q_hbm, msg_hbm,
                 idxd_v, idxs_v, ab_v, gd_v, gs_v, msg_v, sem0, sem1, sem2):
    c = lax.axis_index("c")
    s = lax.axis_index("s")
    w = c * NS + s

    third = jnp.float32(1.0 / 3.0)
    fifth = jnp.float32(1.0 / 5.0)
    seventh = jnp.float32(1.0 / 7.0)

    pltpu.sync_copy(dst_hbm.at[pl.ds(w * EW, EW)], idxd_v)
    pltpu.sync_copy(src_hbm.at[pl.ds(w * EW, EW)], idxs_v)

    def chunk(j, carry):
        base = w * EW + j * CA
        cpq = pltpu.async_copy(q_hbm.at[pl.ds(base, CA)], ab_v, sem2)
        cpd = pltpu.async_copy(dtbl_hbm.at[idxd_v.at[pl.ds(j * CA, CA)]],
                               gd_v, sem0)
        cps = pltpu.async_copy(stbl_hbm.at[idxs_v.at[pl.ds(j * CA, CA)]],
                               gs_v, sem1)
        cpq.wait()
        cpd.wait()
        cps.wait()

        @plsc.parallel_loop(0, CA, unroll=4)
        def edge(e):
            for g in range(HH // 16):
                sl = pl.ds(g * 16, 16)
                slc = pl.ds(HH + g * 16, 16)
                a = ab_v[e, sl] + gd_v[e, sl] + gs_v[e, sl]
                b = ab_v[e, slc] + gd_v[e, slc] + gs_v[e, slc]
                ta = jnp.exp(-jnp.abs(a))
                u = ta / (2.0 + ta)
                u2 = u * u
                poly = 1.0 + u2 * (third + u2 * (fifth + u2 * seventh))
                sp = jnp.maximum(a, 0.0) + 2.0 * u * poly
                tb = jnp.exp(-jnp.abs(b))
                sg = jnp.where(b >= 0.0, 1.0, tb) / (1.0 + tb)
                msg_v[e, sl] = sp * sg
        pltpu.sync_copy(msg_v, msg_hbm.at[pl.ds(base, CA)])
        return carry

    lax.fori_loop(0, NCHA, chunk, 0)


@functools.lru_cache(maxsize=1)
def _get_sc_msg():
  return pl.kernel(
    _sc_msg_body,
    out_type=jax.ShapeDtypeStruct((EPAD, HH), jnp.float32),
    mesh=plsc.VectorSubcoreMesh(core_axis_name="c", subcore_axis_name="s",
                                num_cores=NC, num_subcores=NS),
    scratch_types=[
        pltpu.VMEM((EW,), jnp.int32),
        pltpu.VMEM((EW,), jnp.int32),
        pltpu.VMEM((CA, 2 * HH), jnp.float32),
        pltpu.VMEM((CA, 2 * HH), jnp.float32),
        pltpu.VMEM((CA, 2 * HH), jnp.float32),
        pltpu.VMEM((CA, HH), jnp.float32),
        pltpu.SemaphoreType.DMA,
        pltpu.SemaphoreType.DMA,
        pltpu.SemaphoreType.DMA,
    ],
  )


def _sc_agg_body(list_hbm, cnt_hbm, msg_hbm, agg_hbm,
                 agg_v, lbuf_v, rows_v, cbuf_v, sem0):
    c = lax.axis_index("c")
    s = lax.axis_index("s")
    w = c * NS + s
    zeros16 = jnp.zeros((16,), jnp.float32)

    def zr(i, carry):
        agg_v[pl.ds(i * 16, 16)] = zeros16
        return carry

    lax.fori_loop(0, SRP * HH // 16, zr, 0)

    pltpu.sync_copy(cnt_hbm.at[w], cbuf_v)
    cnt = cbuf_v[pl.ds(0, 16)][0]
    nblk = (cnt + 2047) // 2048

    def block(b, carry):
        pltpu.sync_copy(list_hbm.at[w, pl.ds(b * 2048, 2048)], lbuf_v)
        ngrp = jnp.minimum(128, (cnt - b * 2048 + 15) // 16)

        def group(g, gcarry):
            lv = lbuf_v[pl.ds(g * 16, 16)]
            eids = lax.shift_right_logical(lv, 9)
            dls = jnp.bitwise_and(lv, 511)
            pltpu.async_copy(msg_hbm.at[eids], rows_v, sem0).wait()
            for k in range(16):
                dl = dls[k]
                for g8 in range(HH // 16):
                    sl = pl.ds(dl * HH + g8 * 16, 16)
                    agg_v[sl] = agg_v[sl] + rows_v[k, pl.ds(g8 * 16, 16)]
            return gcarry

        lax.fori_loop(0, ngrp, group, 0)
        return carry

    lax.fori_loop(0, nblk, block, 0)

    def cpo(t, carry):
        for r in range(16):
            row = t * 16 + r
            for g8 in range(HH // 16):
                rows_v[r, pl.ds(g8 * 16, 16)] = agg_v[pl.ds(row * HH + g8 * 16, 16)]
        pltpu.sync_copy(rows_v, agg_hbm.at[pl.ds(w * SR + t * 16, 16)])
        return carry

    lax.fori_loop(0, SR // 16, cpo, 0)


@functools.lru_cache(maxsize=1)
def _get_sc_agg():
  return pl.kernel(
    _sc_agg_body,
    out_type=jax.ShapeDtypeStruct((NP, HH), jnp.float32),
    mesh=plsc.VectorSubcoreMesh(core_axis_name="c", subcore_axis_name="s",
                                num_cores=NC, num_subcores=NS),
    scratch_types=[
        pltpu.VMEM((SRP * HH,), jnp.float32),
        pltpu.VMEM((2048,), jnp.int32),
        pltpu.VMEM((16, HH), jnp.float32),
        pltpu.VMEM((16,), jnp.int32),
        pltpu.SemaphoreType.DMA,
    ],
  )


def kernel(x, edge_index, edge_attr, batch, W_pre_node, b_pre_node,
           W_pre_edge, b_pre_edge, W_soft, b_soft, W_conv, b_conv, gamma,
           beta, W_prepool, b_prepool, W_post, b_post, W_out, b_out):
    dst = edge_index[1]
    src = edge_index[0]
    dst_p = jnp.pad(dst, (0, EE2 - EE))
    src_p = jnp.pad(src, (0, EE2 - EE))
    ea_p = jnp.pad(edge_attr, ((0, EE2 - EE), (0, 0)))

    wd = [jnp.concatenate([W_soft[l, :HH, :], W_conv[l, :HH, :]], axis=1)
          for l in range(LL)]
    ws = [jnp.concatenate([W_soft[l, HH:2 * HH, :], W_conv[l, HH:2 * HH, :]],
                          axis=1) for l in range(LL)]
    wq = jnp.stack([
        jnp.concatenate([W_soft[l, 2 * HH:, :], W_conv[l, 2 * HH:, :]], axis=1)
        for l in range(LL)])
    bq = jnp.stack([
        jnp.concatenate([b_soft[l], b_conv[l]]) for l in range(LL)])

    x_p = jnp.pad(x, ((0, NP - NN), (0, 0)))
    bat_p = jnp.pad(batch, (0, NP - NN), constant_values=BB)

    elist, ecnt = _get_sc_route()(dst)
    h, dtbl, stbl = _prep_h(x_p, W_pre_node, b_pre_node.reshape(1, HH),
                            wd[0], ws[0])
    qm = _prep_q(ea_p, W_pre_edge, b_pre_edge.reshape(1, HH), wq, bq)

    h2 = None
    for l in range(LL):
        msg = _get_sc_msg()(dst_p, src_p, dtbl, stbl, qm[l])
        agg = _get_sc_agg()(elist, ecnt, msg)
        mean, rstd = _stats(agg)
        gam = gamma[l].reshape(1, HH)
        bet = beta[l].reshape(1, HH)
        if l < LL - 1:
            h, dtbl, stbl = _bnproj(agg, h, mean, rstd, gam, bet,
                                    wd[l + 1], ws[l + 1])
        else:
            h2 = _bnfinal(agg, h, mean, rstd, gam, bet, W_prepool,
                          b_prepool.reshape(1, HH))

    wo_pad = jnp.pad(W_out, ((0, 0), (0, HH - 1)))
    bo_pad = jnp.pad(b_out, (0, HH - 1)).reshape(1, HH)
    outp = _post(h2, bat_p.reshape(NP // RB, 1, RB), W_post, b_post,
                 wo_pad, bo_pad)
    return outp[:, 0]


# speedup vs baseline: 1.1482x; 1.0175x over previous
import functools

import jax
import jax.numpy as jnp
from jax import lax
from jax.experimental import pallas as pl
from jax.experimental.pallas import tpu as pltpu
from jax.experimental.pallas import tpu_sc as plsc

NN = 10000
EE = 160000
HH = 128
BB = 64
LL = 3
EPS_ = 1e-5

NC = 2
NS = 16
NW = NC * NS
EE2 = 163840
EW = EE2 // NW
CA = 128
NCHA = EW // CA
NP = 10240
SR = NP // NW
SRP = SR + 8
RCC = 2000
RNCH = EE // RCC
RING = 2080
ECAP = 81 * 2048
EPAD = EE2
TRASH = SR
SENT = EE * 512 + TRASH
RB = 640


def _prep_h_body(x_ref, wn_ref, bn_ref, wd_ref, ws_ref, h_ref, dt_ref, st_ref):
    h = jnp.maximum(
        jnp.dot(x_ref[...], wn_ref[...], preferred_element_type=jnp.float32)
        + bn_ref[...], 0.0)
    h_ref[...] = h
    dt_ref[...] = jnp.dot(h, wd_ref[...], preferred_element_type=jnp.float32)
    st_ref[...] = jnp.dot(h, ws_ref[...], preferred_element_type=jnp.float32)


def _prep_h(x, wn, bn, wd, ws):
    nb = NP // RB
    return pl.pallas_call(
        _prep_h_body,
        grid=(nb,),
        in_specs=[
            pl.BlockSpec((RB, HH), lambda i: (i, 0)),
            pl.BlockSpec((HH, HH), lambda i: (0, 0)),
            pl.BlockSpec((1, HH), lambda i: (0, 0)),
            pl.BlockSpec((HH, 2 * HH), lambda i: (0, 0)),
            pl.BlockSpec((HH, 2 * HH), lambda i: (0, 0)),
        ],
        out_specs=[
            pl.BlockSpec((RB, HH), lambda i: (i, 0)),
            pl.BlockSpec((RB, 2 * HH), lambda i: (i, 0)),
            pl.BlockSpec((RB, 2 * HH), lambda i: (i, 0)),
        ],
        out_shape=[
            jax.ShapeDtypeStruct((NP, HH), jnp.float32),
            jax.ShapeDtypeStruct((NP, 2 * HH), jnp.float32),
            jax.ShapeDtypeStruct((NP, 2 * HH), jnp.float32),
        ],
    )(x, wn, bn, wd, ws)


def _prep_q_body(ea_ref, wpe_ref, bpe_ref, wq_ref, bq_ref, *q_refs):
    e = jnp.maximum(
        jnp.dot(ea_ref[...], wpe_ref[...], preferred_element_type=jnp.float32)
        + bpe_ref[...], 0.0)
    bq = bq_ref[...]
    for l in range(LL):
        q_refs[l][...] = (
            jnp.dot(e, wq_ref[l], preferred_element_type=jnp.float32)
            + bq[l:l + 1, :])


def _prep_q(ea, wpe, bpe, wq, bq):
    nb = EE2 // RB
    return pl.pallas_call(
        _prep_q_body,
        grid=(nb,),
        in_specs=[
            pl.BlockSpec((RB, 16), lambda i: (i, 0)),
            pl.BlockSpec((16, HH), lambda i: (0, 0)),
            pl.BlockSpec((1, HH), lambda i: (0, 0)),
            pl.BlockSpec((LL, HH, 2 * HH), lambda i: (0, 0, 0)),
            pl.BlockSpec((LL, 2 * HH), lambda i: (0, 0)),
        ],
        out_specs=[pl.BlockSpec((RB, 2 * HH), lambda i: (i, 0))] * LL,
        out_shape=[jax.ShapeDtypeStruct((EE2, 2 * HH), jnp.float32)] * LL,
    )(ea, wpe, bpe, wq, bq)


def _stats_body(agg_ref, mean_ref, rstd_ref, acc1, acc2):
    i = pl.program_id(0)

    @pl.when(i == 0)
    def _():
        acc1[...] = jnp.zeros_like(acc1)
        acc2[...] = jnp.zeros_like(acc2)

    a = agg_ref[...]
    acc1[...] += jnp.sum(a, axis=0, keepdims=True)
    acc2[...] += jnp.sum(a * a, axis=0, keepdims=True)

    @pl.when(i == pl.num_programs(0) - 1)
    def _():
        mu = acc1[...] * (1.0 / NN)
        var = acc2[...] * (1.0 / NN) - mu * mu
        mean_ref[...] = mu
        rstd_ref[...] = lax.rsqrt(var + EPS_)


def _stats(agg):
    nb = NP // RB
    return pl.pallas_call(
        _stats_body,
        grid=(nb,),
        in_specs=[pl.BlockSpec((RB, HH), lambda i: (i, 0))],
        out_specs=[
            pl.BlockSpec((1, HH), lambda i: (0, 0)),
            pl.BlockSpec((1, HH), lambda i: (0, 0)),
        ],
        out_shape=[
            jax.ShapeDtypeStruct((1, HH), jnp.float32),
            jax.ShapeDtypeStruct((1, HH), jnp.float32),
        ],
        scratch_shapes=[pltpu.VMEM((1, HH), jnp.float32)] * 2,
    )(agg)


def _bnproj_body(agg_ref, h_ref, mean_ref, rstd_ref, gam_ref, bet_ref,
                 wd_ref, ws_ref, hn_ref, dt_ref, st_ref):
    hn = ((agg_ref[...] - mean_ref[...]) * rstd_ref[...] * gam_ref[...]
          + bet_ref[...] + h_ref[...])
    hn_ref[...] = hn
    dt_ref[...] = jnp.dot(hn, wd_ref[...], preferred_element_type=jnp.float32)
    st_ref[...] = jnp.dot(hn, ws_ref[...], preferred_element_type=jnp.float32)


def _bnproj(agg, h, mean, rstd, gam, bet, wd, ws):
    nb = NP // RB
    return pl.pallas_call(
        _bnproj_body,
        grid=(nb,),
        in_specs=[
            pl.BlockSpec((RB, HH), lambda i: (i, 0)),
            pl.BlockSpec((RB, HH), lambda i: (i, 0)),
            pl.BlockSpec((1, HH), lambda i: (0, 0)),
            pl.BlockSpec((1, HH), lambda i: (0, 0)),
            pl.BlockSpec((1, HH), lambda i: (0, 0)),
            pl.BlockSpec((1, HH), lambda i: (0, 0)),
            pl.BlockSpec((HH, 2 * HH), lambda i: (0, 0)),
            pl.BlockSpec((HH, 2 * HH), lambda i: (0, 0)),
        ],
        out_specs=[
            pl.BlockSpec((RB, HH), lambda i: (i, 0)),
            pl.BlockSpec((RB, 2 * HH), lambda i: (i, 0)),
            pl.BlockSpec((RB, 2 * HH), lambda i: (i, 0)),
        ],
        out_shape=[
            jax.ShapeDtypeStruct((NP, HH), jnp.float32),
            jax.ShapeDtypeStruct((NP, 2 * HH), jnp.float32),
            jax.ShapeDtypeStruct((NP, 2 * HH), jnp.float32),
        ],
    )(agg, h, mean, rstd, gam, bet, wd, ws)


def _bnfinal_body(agg_ref, h_ref, mean_ref, rstd_ref, gam_ref, bet_ref,
                  wpp_ref, bpp_ref, h2_ref):
    hn = ((agg_ref[...] - mean_ref[...]) * rstd_ref[...] * gam_ref[...]
          + bet_ref[...] + h_ref[...])
    h2_ref[...] = jnp.maximum(
        jnp.dot(hn, wpp_ref[...], preferred_element_type=jnp.float32)
        + bpp_ref[...], 0.0)


def _bnfinal(agg, h, mean, rstd, gam, bet, wpp, bpp):
    nb = NP // RB
    return pl.pallas_call(
        _bnfinal_body,
        grid=(nb,),
        in_specs=[
            pl.BlockSpec((RB, HH), lambda i: (i, 0)),
            pl.BlockSpec((RB, HH), lambda i: (i, 0)),
            pl.BlockSpec((1, HH), lambda i: (0, 0)),
            pl.BlockSpec((1, HH), lambda i: (0, 0)),
            pl.BlockSpec((1, HH), lambda i: (0, 0)),
            pl.BlockSpec((1, HH), lambda i: (0, 0)),
            pl.BlockSpec((HH, HH), lambda i: (0, 0)),
            pl.BlockSpec((1, HH), lambda i: (0, 0)),
        ],
        out_specs=pl.BlockSpec((RB, HH), lambda i: (i, 0)),
        out_shape=jax.ShapeDtypeStruct((NP, HH), jnp.float32),
    )(agg, h, mean, rstd, gam, bet, wpp, bpp)


def _post_body(h2_ref, bat_ref, wp_ref, bp_ref, wo_ref, bo_ref, out_ref,
               pooled, cnt):
    i = pl.program_id(0)

    @pl.when(i == 0)
    def _():
        pooled[...] = jnp.zeros_like(pooled)
        cnt[...] = jnp.zeros_like(cnt)

    bat = bat_ref[...].reshape(1, RB)
    rows = lax.broadcasted_iota(jnp.int32, (BB, RB), 0)
    oh = (rows == bat).astype(jnp.float32)
    pooled[...] += jnp.dot(oh, h2_ref[...], preferred_element_type=jnp.float32)
    cnt[...] += jnp.broadcast_to(jnp.sum(oh, axis=1, keepdims=True), (BB, HH))

    @pl.when(i == pl.num_programs(0) - 1)
    def _():
        g = pooled[...] / jnp.maximum(cnt[...], 1.0)
        bp = bp_ref[...]
        for k in range(2):
            g = jnp.maximum(
                jnp.dot(g, wp_ref[k], preferred_element_type=jnp.float32)
                + bp[k:k + 1, :], 0.0)
        out_ref[...] = (jnp.dot(g, wo_ref[...], preferred_element_type=jnp.float32)
                        + bo_ref[...])


def _post(h2, bat3d, wp, bp, wo_pad, bo_pad):
    nb = NP // RB
    return pl.pallas_call(
        _post_body,
        grid=(nb,),
        in_specs=[
            pl.BlockSpec((RB, HH), lambda i: (i, 0)),
            pl.BlockSpec((1, 1, RB), lambda i: (i, 0, 0)),
            pl.BlockSpec((2, HH, HH), lambda i: (0, 0, 0)),
            pl.BlockSpec((2, HH), lambda i: (0, 0)),
            pl.BlockSpec((HH, HH), lambda i: (0, 0)),
            pl.BlockSpec((1, HH), lambda i: (0, 0)),
        ],
        out_specs=pl.BlockSpec((BB, HH), lambda i: (0, 0)),
        out_shape=jax.ShapeDtypeStruct((BB, HH), jnp.float32),
        scratch_shapes=[pltpu.VMEM((BB, HH), jnp.float32)] * 2,
    )(h2, bat3d, wp, bp, wo_pad, bo_pad)


def _sc_route_body(dst_hbm, list_hbm, cnt_hbm, ring_v, dchunk_v, cbuf_v, sem0):
    c = lax.axis_index("c")
    s = lax.axis_index("s")
    w = c * NS + s
    lo = w * SR
    hi = lo + SR
    iota16 = lax.iota(jnp.int32, 16)
    sent16 = jnp.full((16,), SENT, jnp.int32)

    def chunk(j, carry):
        pltpu.sync_copy(dst_hbm.at[pl.ds(j * RCC, RCC)], dchunk_v)

        def group(g, gcarry):
            goff, gbc = gcarry
            dv = dchunk_v[pl.ds(g * 16, 16)]
            eids = j * RCC + g * 16 + iota16
            m = jnp.logical_and(dv >= lo, dv < hi)
            packed = jnp.where(m, eids * 512 + (dv - lo), sent16)
            mi = jnp.where(m, 1, 0)
            for k in range(16):
                ring_v[pl.ds(goff, 16)] = jnp.full((16,), packed[k], jnp.int32)
                goff = goff + mi[k]

            def flush(args):
                foff, fbc = args
                pltpu.sync_copy(ring_v.at[pl.ds(0, 2048)],
                                list_hbm.at[w, pl.ds(fbc * 2048, 2048)])
                tailv = ring_v[pl.ds(2048, 16)]
                ring_v[pl.ds(0, 16)] = tailv
                return foff - 2048, fbc + 1

            return lax.cond(goff >= 2048, flush, lambda a: a, (goff, gbc))

        return lax.fori_loop(0, RCC // 16, group, carry)

    off, bc = lax.fori_loop(0, RNCH, chunk, (jnp.int32(0), jnp.int32(0)))
    ring_v[pl.ds(off, 16)] = sent16
    off = ((off + 15) // 16) * 16
    pltpu.sync_copy(ring_v.at[pl.ds(0, 2048)],
                    list_hbm.at[w, pl.ds(bc * 2048, 2048)])
    cbuf_v[pl.ds(0, 16)] = jnp.full((16,), bc * 2048 + off, jnp.int32)
    pltpu.sync_copy(cbuf_v, cnt_hbm.at[w])


@functools.lru_cache(maxsize=1)
def _get_sc_route():
  return pl.kernel(
    _sc_route_body,
    out_type=(
        jax.ShapeDtypeStruct((NW, ECAP), jnp.int32),
        jax.ShapeDtypeStruct((NW, 16), jnp.int32),
    ),
    mesh=plsc.VectorSubcoreMesh(core_axis_name="c", subcore_axis_name="s",
                                num_cores=NC, num_subcores=NS),
    scratch_types=[
        pltpu.VMEM((RING,), jnp.int32),
        pltpu.VMEM((RCC,), jnp.int32),
        pltpu.VMEM((16,), jnp.int32),
        pltpu.SemaphoreType.DMA,
    ],
  )


def _sc_msg_body(dst_hbm, src_hbm, dtbl_hbm, stbl_hbm, q_hbm, msg_hbm,
                 idxd_v, idxs_v, ab_v, gd_v, gs_v, msg_v, sem0, sem1, sem2):
    c = lax.axis_index("c")
    s = lax.axis_index("s")
    w = c * NS + s

    c1 = jnp.float32(0.999957466)
    c2 = jnp.float32(-0.499206382)
    c3 = jnp.float32(0.326972352)
    c4 = jnp.float32(-0.222834717)
    c5 = jnp.float32(0.130763359)
    c6 = jnp.float32(-0.0526239552)
    c7 = jnp.float32(0.0101189017)

    pltpu.sync_copy(dst_hbm.at[pl.ds(w * EW, EW)], idxd_v)
    pltpu.sync_copy(src_hbm.at[pl.ds(w * EW, EW)], idxs_v)

    def chunk(j, carry):
        base = w * EW + j * CA
        cpq = pltpu.async_copy(q_hbm.at[pl.ds(base, CA)], ab_v, sem2)
        cpd = pltpu.async_copy(dtbl_hbm.at[idxd_v.at[pl.ds(j * CA, CA)]],
                               gd_v, sem0)
        cps = pltpu.async_copy(stbl_hbm.at[idxs_v.at[pl.ds(j * CA, CA)]],
                               gs_v, sem1)
        cpq.wait()
        cpd.wait()
        cps.wait()

        @plsc.parallel_loop(0, CA, unroll=4)
        def edge(e):
            for g in range(HH // 16):
                sl = pl.ds(g * 16, 16)
                slc = pl.ds(HH + g * 16, 16)
                a = ab_v[e, sl] + gd_v[e, sl] + gs_v[e, sl]
                b = ab_v[e, slc] + gd_v[e, slc] + gs_v[e, slc]
                ta = jnp.exp(-jnp.abs(a))
                poly = c1 + ta * (c2 + ta * (c3 + ta * (c4 + ta * (
                    c5 + ta * (c6 + ta * c7)))))
                sp = jnp.maximum(a, 0.0) + ta * poly
                tb = jnp.exp(-jnp.abs(b))
                num = jnp.where(b >= 0.0, sp, sp * tb)
                msg_v[e, sl] = num / (1.0 + tb)
        pltpu.sync_copy(msg_v, msg_hbm.at[pl.ds(base, CA)])
        return carry

    lax.fori_loop(0, NCHA, chunk, 0)


@functools.lru_cache(maxsize=1)
def _get_sc_msg():
  return pl.kernel(
    _sc_msg_body,
    out_type=jax.ShapeDtypeStruct((EPAD, HH), jnp.float32),
    mesh=plsc.VectorSubcoreMesh(core_axis_name="c", subcore_axis_name="s",
                                num_cores=NC, num_subcores=NS),
    scratch_types=[
        pltpu.VMEM((EW,), jnp.int32),
        pltpu.VMEM((EW,), jnp.int32),
        pltpu.VMEM((CA, 2 * HH), jnp.float32),
        pltpu.VMEM((CA, 2 * HH), jnp.float32),
        pltpu.VMEM((CA, 2 * HH), jnp.float32),
        pltpu.VMEM((CA, HH), jnp.float32),
        pltpu.SemaphoreType.DMA,
        pltpu.SemaphoreType.DMA,
        pltpu.SemaphoreType.DMA,
    ],
  )


def _sc_agg_body(list_hbm, cnt_hbm, msg_hbm, agg_hbm,
                 agg_v, lbuf_v, rows_v, cbuf_v, sem0):
    c = lax.axis_index("c")
    s = lax.axis_index("s")
    w = c * NS + s
    zeros16 = jnp.zeros((16,), jnp.float32)

    def zr(i, carry):
        agg_v[pl.ds(i * 16, 16)] = zeros16
        return carry

    lax.fori_loop(0, SRP * HH // 16, zr, 0)

    pltpu.sync_copy(cnt_hbm.at[w], cbuf_v)
    cnt = cbuf_v[pl.ds(0, 16)][0]
    nblk = (cnt + 2047) // 2048

    def block(b, carry):
        pltpu.sync_copy(list_hbm.at[w, pl.ds(b * 2048, 2048)], lbuf_v)
        ngrp = jnp.minimum(128, (cnt - b * 2048 + 15) // 16)

        def group(g, gcarry):
            lv = lbuf_v[pl.ds(g * 16, 16)]
            eids = lax.shift_right_logical(lv, 9)
            dls = jnp.bitwise_and(lv, 511)
            pltpu.async_copy(msg_hbm.at[eids], rows_v, sem0).wait()
            for k in range(16):
                dl = dls[k]
                for g8 in range(HH // 16):
                    sl = pl.ds(dl * HH + g8 * 16, 16)
                    agg_v[sl] = agg_v[sl] + rows_v[k, pl.ds(g8 * 16, 16)]
            return gcarry

        lax.fori_loop(0, ngrp, group, 0)
        return carry

    lax.fori_loop(0, nblk, block, 0)

    def cpo(t, carry):
        for r in range(16):
            row = t * 16 + r
            for g8 in range(HH // 16):
                rows_v[r, pl.ds(g8 * 16, 16)] = agg_v[pl.ds(row * HH + g8 * 16, 16)]
        pltpu.sync_copy(rows_v, agg_hbm.at[pl.ds(w * SR + t * 16, 16)])
        return carry

    lax.fori_loop(0, SR // 16, cpo, 0)


@functools.lru_cache(maxsize=1)
def _get_sc_agg():
  return pl.kernel(
    _sc_agg_body,
    out_type=jax.ShapeDtypeStruct((NP, HH), jnp.float32),
    mesh=plsc.VectorSubcoreMesh(core_axis_name="c", subcore_axis_name="s",
                                num_cores=NC, num_subcores=NS),
    scratch_types=[
        pltpu.VMEM((SRP * HH,), jnp.float32),
        pltpu.VMEM((2048,), jnp.int32),
        pltpu.VMEM((16, HH), jnp.float32),
        pltpu.VMEM((16,), jnp.int32),
        pltpu.SemaphoreType.DMA,
    ],
  )


def kernel(x, edge_index, edge_attr, batch, W_pre_node, b_pre_node,
           W_pre_edge, b_pre_edge, W_soft, b_soft, W_conv, b_conv, gamma,
           beta, W_prepool, b_prepool, W_post, b_post, W_out, b_out):
    dst = edge_index[1]
    src = edge_index[0]
    dst_p = jnp.pad(dst, (0, EE2 - EE))
    src_p = jnp.pad(src, (0, EE2 - EE))
    ea_p = jnp.pad(edge_attr, ((0, EE2 - EE), (0, 0)))

    wd = [jnp.concatenate([W_soft[l, :HH, :], W_conv[l, :HH, :]], axis=1)
          for l in range(LL)]
    ws = [jnp.concatenate([W_soft[l, HH:2 * HH, :], W_conv[l, HH:2 * HH, :]],
                          axis=1) for l in range(LL)]
    wq = jnp.stack([
        jnp.concatenate([W_soft[l, 2 * HH:, :], W_conv[l, 2 * HH:, :]], axis=1)
        for l in range(LL)])
    bq = jnp.stack([
        jnp.concatenate([b_soft[l], b_conv[l]]) for l in range(LL)])

    x_p = jnp.pad(x, ((0, NP - NN), (0, 0)))
    bat_p = jnp.pad(batch, (0, NP - NN), constant_values=BB)

    elist, ecnt = _get_sc_route()(dst)
    h, dtbl, stbl = _prep_h(x_p, W_pre_node, b_pre_node.reshape(1, HH),
                            wd[0], ws[0])
    qm = _prep_q(ea_p, W_pre_edge, b_pre_edge.reshape(1, HH), wq, bq)

    h2 = None
    for l in range(LL):
        msg = _get_sc_msg()(dst_p, src_p, dtbl, stbl, qm[l])
        agg = _get_sc_agg()(elist, ecnt, msg)
        mean, rstd = _stats(agg)
        gam = gamma[l].reshape(1, HH)
        bet = beta[l].reshape(1, HH)
        if l < LL - 1:
            h, dtbl, stbl = _bnproj(agg, h, mean, rstd, gam, bet,
                                    wd[l + 1], ws[l + 1])
        else:
            h2 = _bnfinal(agg, h, mean, rstd, gam, bet, W_prepool,
                          b_prepool.reshape(1, HH))

    wo_pad = jnp.pad(W_out, ((0, 0), (0, HH - 1)))
    bo_pad = jnp.pad(b_out, (0, HH - 1)).reshape(1, HH)
    outp = _post(h2, bat_p.reshape(NP // RB, 1, RB), W_post, b_post,
                 wo_pad, bo_pad)
    return outp[:, 0]


# speedup vs baseline: 1.4597x; 1.2714x over previous
import functools

import jax
import jax.numpy as jnp
from jax import lax
from jax.experimental import pallas as pl
from jax.experimental.pallas import tpu as pltpu
from jax.experimental.pallas import tpu_sc as plsc

NN = 10000
EE = 160000
HH = 128
BB = 64
LL = 3
EPS_ = 1e-5

NC = 2
NS = 16
NW = NC * NS
EE2 = 163840
EW = EE2 // NW
CA = 128
NCHA = EW // CA
NP = 10240
SR = NP // NW
SRP = SR + 8
RCC = 2000
RNCH = EE // RCC
RING = 2080
ECAP = 81 * 2048
EPAD = EE2
TRASH = SR
SENT = EE * 512 + TRASH
RB = 640


def _prep_h_body(x_ref, wn_ref, bn_ref, wd_ref, ws_ref, h_ref, dt_ref, st_ref):
    h = jnp.maximum(
        jnp.dot(x_ref[...], wn_ref[...], preferred_element_type=jnp.float32)
        + bn_ref[...], 0.0)
    h_ref[...] = h
    dt_ref[...] = jnp.dot(h, wd_ref[...], preferred_element_type=jnp.float32)
    st_ref[...] = jnp.dot(h, ws_ref[...], preferred_element_type=jnp.float32)


def _prep_h(x, wn, bn, wd, ws):
    nb = NP // RB
    return pl.pallas_call(
        _prep_h_body,
        grid=(nb,),
        in_specs=[
            pl.BlockSpec((RB, HH), lambda i: (i, 0)),
            pl.BlockSpec((HH, HH), lambda i: (0, 0)),
            pl.BlockSpec((1, HH), lambda i: (0, 0)),
            pl.BlockSpec((HH, 2 * HH), lambda i: (0, 0)),
            pl.BlockSpec((HH, 2 * HH), lambda i: (0, 0)),
        ],
        out_specs=[
            pl.BlockSpec((RB, HH), lambda i: (i, 0)),
            pl.BlockSpec((RB, 2 * HH), lambda i: (i, 0)),
            pl.BlockSpec((RB, 2 * HH), lambda i: (i, 0)),
        ],
        out_shape=[
            jax.ShapeDtypeStruct((NP, HH), jnp.float32),
            jax.ShapeDtypeStruct((NP, 2 * HH), jnp.float32),
            jax.ShapeDtypeStruct((NP, 2 * HH), jnp.float32),
        ],
    )(x, wn, bn, wd, ws)


def _prep_q_body(ea_ref, wpe_ref, bpe_ref, wq_ref, bq_ref, *q_refs):
    e = jnp.maximum(
        jnp.dot(ea_ref[...], wpe_ref[...], preferred_element_type=jnp.float32)
        + bpe_ref[...], 0.0)
    bq = bq_ref[...]
    for l in range(LL):
        q_refs[l][...] = (
            jnp.dot(e, wq_ref[l], preferred_element_type=jnp.float32)
            + bq[l:l + 1, :])


def _prep_q(ea, wpe, bpe, wq, bq):
    nb = EE2 // RB
    return pl.pallas_call(
        _prep_q_body,
        grid=(nb,),
        in_specs=[
            pl.BlockSpec((RB, 16), lambda i: (i, 0)),
            pl.BlockSpec((16, HH), lambda i: (0, 0)),
            pl.BlockSpec((1, HH), lambda i: (0, 0)),
            pl.BlockSpec((LL, HH, 2 * HH), lambda i: (0, 0, 0)),
            pl.BlockSpec((LL, 2 * HH), lambda i: (0, 0)),
        ],
        out_specs=[pl.BlockSpec((RB, 2 * HH), lambda i: (i, 0))] * LL,
        out_shape=[jax.ShapeDtypeStruct((EE2, 2 * HH), jnp.float32)] * LL,
    )(ea, wpe, bpe, wq, bq)


def _stats_body(agg_ref, mean_ref, rstd_ref, acc1, acc2):
    i = pl.program_id(0)

    @pl.when(i == 0)
    def _():
        acc1[...] = jnp.zeros_like(acc1)
        acc2[...] = jnp.zeros_like(acc2)

    a = agg_ref[...]
    acc1[...] += jnp.sum(a, axis=0, keepdims=True)
    acc2[...] += jnp.sum(a * a, axis=0, keepdims=True)

    @pl.when(i == pl.num_programs(0) - 1)
    def _():
        mu = acc1[...] * (1.0 / NN)
        var = acc2[...] * (1.0 / NN) - mu * mu
        mean_ref[...] = mu
        rstd_ref[...] = lax.rsqrt(var + EPS_)


def _stats(agg):
    nb = NP // RB
    return pl.pallas_call(
        _stats_body,
        grid=(nb,),
        in_specs=[pl.BlockSpec((RB, HH), lambda i: (i, 0))],
        out_specs=[
            pl.BlockSpec((1, HH), lambda i: (0, 0)),
            pl.BlockSpec((1, HH), lambda i: (0, 0)),
        ],
        out_shape=[
            jax.ShapeDtypeStruct((1, HH), jnp.float32),
            jax.ShapeDtypeStruct((1, HH), jnp.float32),
        ],
        scratch_shapes=[pltpu.VMEM((1, HH), jnp.float32)] * 2,
    )(agg)


def _bnproj_body(agg_ref, h_ref, mean_ref, rstd_ref, gam_ref, bet_ref,
                 wd_ref, ws_ref, hn_ref, dt_ref, st_ref):
    hn = ((agg_ref[...] - mean_ref[...]) * rstd_ref[...] * gam_ref[...]
          + bet_ref[...] + h_ref[...])
    hn_ref[...] = hn
    dt_ref[...] = jnp.dot(hn, wd_ref[...], preferred_element_type=jnp.float32)
    st_ref[...] = jnp.dot(hn, ws_ref[...], preferred_element_type=jnp.float32)


def _bnproj(agg, h, mean, rstd, gam, bet, wd, ws):
    nb = NP // RB
    return pl.pallas_call(
        _bnproj_body,
        grid=(nb,),
        in_specs=[
            pl.BlockSpec((RB, HH), lambda i: (i, 0)),
            pl.BlockSpec((RB, HH), lambda i: (i, 0)),
            pl.BlockSpec((1, HH), lambda i: (0, 0)),
            pl.BlockSpec((1, HH), lambda i: (0, 0)),
            pl.BlockSpec((1, HH), lambda i: (0, 0)),
            pl.BlockSpec((1, HH), lambda i: (0, 0)),
            pl.BlockSpec((HH, 2 * HH), lambda i: (0, 0)),
            pl.BlockSpec((HH, 2 * HH), lambda i: (0, 0)),
        ],
        out_specs=[
            pl.BlockSpec((RB, HH), lambda i: (i, 0)),
            pl.BlockSpec((RB, 2 * HH), lambda i: (i, 0)),
            pl.BlockSpec((RB, 2 * HH), lambda i: (i, 0)),
        ],
        out_shape=[
            jax.ShapeDtypeStruct((NP, HH), jnp.float32),
            jax.ShapeDtypeStruct((NP, 2 * HH), jnp.float32),
            jax.ShapeDtypeStruct((NP, 2 * HH), jnp.float32),
        ],
    )(agg, h, mean, rstd, gam, bet, wd, ws)


def _bnfinal_body(agg_ref, h_ref, mean_ref, rstd_ref, gam_ref, bet_ref,
                  wpp_ref, bpp_ref, h2_ref):
    hn = ((agg_ref[...] - mean_ref[...]) * rstd_ref[...] * gam_ref[...]
          + bet_ref[...] + h_ref[...])
    h2_ref[...] = jnp.maximum(
        jnp.dot(hn, wpp_ref[...], preferred_element_type=jnp.float32)
        + bpp_ref[...], 0.0)


def _bnfinal(agg, h, mean, rstd, gam, bet, wpp, bpp):
    nb = NP // RB
    return pl.pallas_call(
        _bnfinal_body,
        grid=(nb,),
        in_specs=[
            pl.BlockSpec((RB, HH), lambda i: (i, 0)),
            pl.BlockSpec((RB, HH), lambda i: (i, 0)),
            pl.BlockSpec((1, HH), lambda i: (0, 0)),
            pl.BlockSpec((1, HH), lambda i: (0, 0)),
            pl.BlockSpec((1, HH), lambda i: (0, 0)),
            pl.BlockSpec((1, HH), lambda i: (0, 0)),
            pl.BlockSpec((HH, HH), lambda i: (0, 0)),
            pl.BlockSpec((1, HH), lambda i: (0, 0)),
        ],
        out_specs=pl.BlockSpec((RB, HH), lambda i: (i, 0)),
        out_shape=jax.ShapeDtypeStruct((NP, HH), jnp.float32),
    )(agg, h, mean, rstd, gam, bet, wpp, bpp)


def _post_body(h2_ref, bat_ref, wp_ref, bp_ref, wo_ref, bo_ref, out_ref,
               pooled, cnt):
    i = pl.program_id(0)

    @pl.when(i == 0)
    def _():
        pooled[...] = jnp.zeros_like(pooled)
        cnt[...] = jnp.zeros_like(cnt)

    bat = bat_ref[...].reshape(1, RB)
    rows = lax.broadcasted_iota(jnp.int32, (BB, RB), 0)
    oh = (rows == bat).astype(jnp.float32)
    pooled[...] += jnp.dot(oh, h2_ref[...], preferred_element_type=jnp.float32)
    cnt[...] += jnp.broadcast_to(jnp.sum(oh, axis=1, keepdims=True), (BB, HH))

    @pl.when(i == pl.num_programs(0) - 1)
    def _():
        g = pooled[...] / jnp.maximum(cnt[...], 1.0)
        bp = bp_ref[...]
        for k in range(2):
            g = jnp.maximum(
                jnp.dot(g, wp_ref[k], preferred_element_type=jnp.float32)
                + bp[k:k + 1, :], 0.0)
        out_ref[...] = (jnp.dot(g, wo_ref[...], preferred_element_type=jnp.float32)
                        + bo_ref[...])


def _post(h2, bat3d, wp, bp, wo_pad, bo_pad):
    nb = NP // RB
    return pl.pallas_call(
        _post_body,
        grid=(nb,),
        in_specs=[
            pl.BlockSpec((RB, HH), lambda i: (i, 0)),
            pl.BlockSpec((1, 1, RB), lambda i: (i, 0, 0)),
            pl.BlockSpec((2, HH, HH), lambda i: (0, 0, 0)),
            pl.BlockSpec((2, HH), lambda i: (0, 0)),
            pl.BlockSpec((HH, HH), lambda i: (0, 0)),
            pl.BlockSpec((1, HH), lambda i: (0, 0)),
        ],
        out_specs=pl.BlockSpec((BB, HH), lambda i: (0, 0)),
        out_shape=jax.ShapeDtypeStruct((BB, HH), jnp.float32),
        scratch_shapes=[pltpu.VMEM((BB, HH), jnp.float32)] * 2,
    )(h2, bat3d, wp, bp, wo_pad, bo_pad)


def _sc_route_body(dst_hbm, list_hbm, cnt_hbm, ring_v, dchunk_v, cbuf_v, sem0):
    c = lax.axis_index("c")
    s = lax.axis_index("s")
    w = c * NS + s
    lo = w * SR
    hi = lo + SR
    iota16 = lax.iota(jnp.int32, 16)
    sent16 = jnp.full((16,), SENT, jnp.int32)

    def chunk(j, carry):
        pltpu.sync_copy(dst_hbm.at[pl.ds(j * RCC, RCC)], dchunk_v)

        def group(g, gcarry):
            goff, gbc = gcarry
            dv = dchunk_v[pl.ds(g * 16, 16)]
            eids = j * RCC + g * 16 + iota16
            m = jnp.logical_and(dv >= lo, dv < hi)
            packed = jnp.where(m, eids * 512 + (dv - lo), sent16)
            mi = jnp.where(m, 1, 0)
            for k in range(16):
                ring_v[pl.ds(goff, 16)] = jnp.full((16,), packed[k], jnp.int32)
                goff = goff + mi[k]

            def flush(args):
                foff, fbc = args
                pltpu.sync_copy(ring_v.at[pl.ds(0, 2048)],
                                list_hbm.at[w, pl.ds(fbc * 2048, 2048)])
                tailv = ring_v[pl.ds(2048, 16)]
                ring_v[pl.ds(0, 16)] = tailv
                return foff - 2048, fbc + 1

            return lax.cond(goff >= 2048, flush, lambda a: a, (goff, gbc))

        return lax.fori_loop(0, RCC // 16, group, carry)

    off, bc = lax.fori_loop(0, RNCH, chunk, (jnp.int32(0), jnp.int32(0)))
    ring_v[pl.ds(off, 16)] = sent16
    off = ((off + 15) // 16) * 16
    pltpu.sync_copy(ring_v.at[pl.ds(0, 2048)],
                    list_hbm.at[w, pl.ds(bc * 2048, 2048)])
    cbuf_v[pl.ds(0, 16)] = jnp.full((16,), bc * 2048 + off, jnp.int32)
    pltpu.sync_copy(cbuf_v, cnt_hbm.at[w])


@functools.lru_cache(maxsize=1)
def _get_sc_route():
  return pl.kernel(
    _sc_route_body,
    out_type=(
        jax.ShapeDtypeStruct((NW, ECAP), jnp.int32),
        jax.ShapeDtypeStruct((NW, 16), jnp.int32),
    ),
    mesh=plsc.VectorSubcoreMesh(core_axis_name="c", subcore_axis_name="s",
                                num_cores=NC, num_subcores=NS),
    scratch_types=[
        pltpu.VMEM((RING,), jnp.int32),
        pltpu.VMEM((RCC,), jnp.int32),
        pltpu.VMEM((16,), jnp.int32),
        pltpu.SemaphoreType.DMA,
    ],
  )


def _sc_msg_body(dst_hbm, src_hbm, dtbl_hbm, stbl_hbm, q_hbm, msg_hbm,
                 idxd_v, idxs_v, ab_v, gd_v, gs_v, msg_v, sem0, sem1, sem2):
    c = lax.axis_index("c")
    s = lax.axis_index("s")
    w = c * NS + s

    c1 = jnp.float32(0.999957466)
    c2 = jnp.float32(-0.499206382)
    c3 = jnp.float32(0.326972352)
    c4 = jnp.float32(-0.222834717)
    c5 = jnp.float32(0.130763359)
    c6 = jnp.float32(-0.0526239552)
    c7 = jnp.float32(0.0101189017)

    pltpu.sync_copy(dst_hbm.at[pl.ds(w * EW, EW)], idxd_v)
    pltpu.sync_copy(src_hbm.at[pl.ds(w * EW, EW)], idxs_v)

    def chunk(j, carry):
        base = w * EW + j * CA
        cpq = pltpu.async_copy(q_hbm.at[pl.ds(base, CA)], ab_v, sem2)
        cpd = pltpu.async_copy(dtbl_hbm.at[idxd_v.at[pl.ds(j * CA, CA)]],
                               gd_v, sem0)
        cps = pltpu.async_copy(stbl_hbm.at[idxs_v.at[pl.ds(j * CA, CA)]],
                               gs_v, sem1)
        cpq.wait()
        cpd.wait()
        cps.wait()

        mrnd = jnp.float32(12582912.0)
        nl2e = jnp.float32(-1.4426950408889634)
        e0 = jnp.float32(1.00000005)
        e1 = jnp.float32(0.6931472)
        e2 = jnp.float32(0.240222117)
        e3 = jnp.float32(0.0555034068)
        e4 = jnp.float32(0.00967076313)
        e5 = jnp.float32(0.00133952798)

        def expneg(t):
            u = t * nl2e
            kf = (u + mrnd) - mrnd
            r = u - kf
            ki = lax.convert_element_type(kf, jnp.int32)
            e2k = lax.bitcast_convert_type(lax.shift_left(ki + 127, 23), jnp.float32)
            p = e0 + r * (e1 + r * (e2 + r * (e3 + r * (e4 + r * e5))))
            return p * e2k

        @plsc.parallel_loop(0, CA, unroll=4)
        def edge(e):
            for g in range(HH // 16):
                sl = pl.ds(g * 16, 16)
                slc = pl.ds(HH + g * 16, 16)
                a = ab_v[e, sl] + gd_v[e, sl] + gs_v[e, sl]
                b = ab_v[e, slc] + gd_v[e, slc] + gs_v[e, slc]
                ta = expneg(jnp.minimum(jnp.abs(a), 25.0))
                poly = c1 + ta * (c2 + ta * (c3 + ta * (c4 + ta * (
                    c5 + ta * (c6 + ta * c7)))))
                sp = jnp.maximum(a, 0.0) + ta * poly
                tb = expneg(jnp.minimum(jnp.abs(b), 25.0))
                num = jnp.where(b >= 0.0, sp, sp * tb)
                d = 1.0 + tb
                y = jnp.float32(48.0 / 17.0) + d * jnp.float32(-32.0 / 17.0)
                y = y * (2.0 - d * y)
                y = y * (2.0 - d * y)
                msg_v[e, sl] = num * y
        pltpu.sync_copy(msg_v, msg_hbm.at[pl.ds(base, CA)])
        return carry

    lax.fori_loop(0, NCHA, chunk, 0)


@functools.lru_cache(maxsize=1)
def _get_sc_msg():
  return pl.kernel(
    _sc_msg_body,
    out_type=jax.ShapeDtypeStruct((EPAD, HH), jnp.float32),
    mesh=plsc.VectorSubcoreMesh(core_axis_name="c", subcore_axis_name="s",
                                num_cores=NC, num_subcores=NS),
    scratch_types=[
        pltpu.VMEM((EW,), jnp.int32),
        pltpu.VMEM((EW,), jnp.int32),
        pltpu.VMEM((CA, 2 * HH), jnp.float32),
        pltpu.VMEM((CA, 2 * HH), jnp.float32),
        pltpu.VMEM((CA, 2 * HH), jnp.float32),
        pltpu.VMEM((CA, HH), jnp.float32),
        pltpu.SemaphoreType.DMA,
        pltpu.SemaphoreType.DMA,
        pltpu.SemaphoreType.DMA,
    ],
  )


def _sc_agg_body(list_hbm, cnt_hbm, msg_hbm, agg_hbm,
                 agg_v, lbuf_v, rows_v, cbuf_v, sem0):
    c = lax.axis_index("c")
    s = lax.axis_index("s")
    w = c * NS + s
    zeros16 = jnp.zeros((16,), jnp.float32)

    def zr(i, carry):
        agg_v[pl.ds(i * 16, 16)] = zeros16
        return carry

    lax.fori_loop(0, SRP * HH // 16, zr, 0)

    pltpu.sync_copy(cnt_hbm.at[w], cbuf_v)
    cnt = cbuf_v[pl.ds(0, 16)][0]
    nblk = (cnt + 2047) // 2048

    def block(b, carry):
        pltpu.sync_copy(list_hbm.at[w, pl.ds(b * 2048, 2048)], lbuf_v)
        ngrp = jnp.minimum(128, (cnt - b * 2048 + 15) // 16)

        def group(g, gcarry):
            lv = lbuf_v[pl.ds(g * 16, 16)]
            eids = lax.shift_right_logical(lv, 9)
            dls = jnp.bitwise_and(lv, 511)
            pltpu.async_copy(msg_hbm.at[eids], rows_v, sem0).wait()
            for k in range(16):
                dl = dls[k]
                for g8 in range(HH // 16):
                    sl = pl.ds(dl * HH + g8 * 16, 16)
                    agg_v[sl] = agg_v[sl] + rows_v[k, pl.ds(g8 * 16, 16)]
            return gcarry

        lax.fori_loop(0, ngrp, group, 0)
        return carry

    lax.fori_loop(0, nblk, block, 0)

    def cpo(t, carry):
        for r in range(16):
            row = t * 16 + r
            for g8 in range(HH // 16):
                rows_v[r, pl.ds(g8 * 16, 16)] = agg_v[pl.ds(row * HH + g8 * 16, 16)]
        pltpu.sync_copy(rows_v, agg_hbm.at[pl.ds(w * SR + t * 16, 16)])
        return carry

    lax.fori_loop(0, SR // 16, cpo, 0)


@functools.lru_cache(maxsize=1)
def _get_sc_agg():
  return pl.kernel(
    _sc_agg_body,
    out_type=jax.ShapeDtypeStruct((NP, HH), jnp.float32),
    mesh=plsc.VectorSubcoreMesh(core_axis_name="c", subcore_axis_name="s",
                                num_cores=NC, num_subcores=NS),
    scratch_types=[
        pltpu.VMEM((SRP * HH,), jnp.float32),
        pltpu.VMEM((2048,), jnp.int32),
        pltpu.VMEM((16, HH), jnp.float32),
        pltpu.VMEM((16,), jnp.int32),
        pltpu.SemaphoreType.DMA,
    ],
  )


def kernel(x, edge_index, edge_attr, batch, W_pre_node, b_pre_node,
           W_pre_edge, b_pre_edge, W_soft, b_soft, W_conv, b_conv, gamma,
           beta, W_prepool, b_prepool, W_post, b_post, W_out, b_out):
    dst = edge_index[1]
    src = edge_index[0]
    dst_p = jnp.pad(dst, (0, EE2 - EE))
    src_p = jnp.pad(src, (0, EE2 - EE))
    ea_p = jnp.pad(edge_attr, ((0, EE2 - EE), (0, 0)))

    wd = [jnp.concatenate([W_soft[l, :HH, :], W_conv[l, :HH, :]], axis=1)
          for l in range(LL)]
    ws = [jnp.concatenate([W_soft[l, HH:2 * HH, :], W_conv[l, HH:2 * HH, :]],
                          axis=1) for l in range(LL)]
    wq = jnp.stack([
        jnp.concatenate([W_soft[l, 2 * HH:, :], W_conv[l, 2 * HH:, :]], axis=1)
        for l in range(LL)])
    bq = jnp.stack([
        jnp.concatenate([b_soft[l], b_conv[l]]) for l in range(LL)])

    x_p = jnp.pad(x, ((0, NP - NN), (0, 0)))
    bat_p = jnp.pad(batch, (0, NP - NN), constant_values=BB)

    elist, ecnt = _get_sc_route()(dst)
    h, dtbl, stbl = _prep_h(x_p, W_pre_node, b_pre_node.reshape(1, HH),
                            wd[0], ws[0])
    qm = _prep_q(ea_p, W_pre_edge, b_pre_edge.reshape(1, HH), wq, bq)

    h2 = None
    for l in range(LL):
        msg = _get_sc_msg()(dst_p, src_p, dtbl, stbl, qm[l])
        agg = _get_sc_agg()(elist, ecnt, msg)
        mean, rstd = _stats(agg)
        gam = gamma[l].reshape(1, HH)
        bet = beta[l].reshape(1, HH)
        if l < LL - 1:
            h, dtbl, stbl = _bnproj(agg, h, mean, rstd, gam, bet,
                                    wd[l + 1], ws[l + 1])
        else:
            h2 = _bnfinal(agg, h, mean, rstd, gam, bet, W_prepool,
                          b_prepool.reshape(1, HH))

    wo_pad = jnp.pad(W_out, ((0, 0), (0, HH - 1)))
    bo_pad = jnp.pad(b_out, (0, HH - 1)).reshape(1, HH)
    outp = _post(h2, bat_p.reshape(NP // RB, 1, RB), W_post, b_post,
                 wo_pad, bo_pad)
    return outp[:, 0]
